# Initial kernel scaffold; baseline (speedup 1.0000x reference)
import jax
import jax.numpy as jnp
from jax.experimental import pallas as pl


def kernel(in_feat, edge_index, W1, b1, W2, b2):
    raise NotImplementedError("write your pallas kernel here")


# speedup vs baseline: 5.1258x; 5.1258x over previous
import functools

import jax
import jax.numpy as jnp
from jax import lax
from jax.experimental import pallas as pl
from jax.experimental.pallas import tpu as pltpu
from jax.experimental.pallas import tpu_sc as plsc

N = 10000
NP = 10240
E = 320000
D = 128
H = 128
C = 16

NC = 2
NS = 16
LANES = 16

_SC_PARAMS = pltpu.CompilerParams(use_tc_tiling_on_sc=False)


def _mesh():
    return plsc.VectorSubcoreMesh(core_axis_name="c", subcore_axis_name="s")


_K1_CHUNK = 128
_K1_EPT = E // NS
_K1_FULL = _K1_EPT // _K1_CHUNK
_K1_TAIL = _K1_EPT - _K1_FULL * _K1_CHUNK
_K1_CMB = 656
_K1_PRIV = _K1_CMB * NS
_K1_LAST = NP - (NS - 1) * _K1_CMB


def _deg_body(edges_hbm, degs_hbm, idx2, ones_r, dbuf, cbuf, parts):
    c = lax.axis_index("c")
    s = lax.axis_index("s")

    zv = jnp.zeros((LANES,), jnp.float32)
    ov = jnp.ones((LANES,), jnp.float32)

    def fo(r, _):
        ones_r[pl.ds(r * LANES, LANES)] = ov
        return 0

    lax.fori_loop(0, _K1_CHUNK // LANES, fo, 0)

    def fz(r, _):
        dbuf[pl.ds(r * LANES, LANES)] = zv
        return 0

    lax.fori_loop(0, _K1_CMB // LANES, fz, 0)

    for t in range(NS):
        pltpu.sync_copy(dbuf,
                        parts.at[pl.ds(s * _K1_PRIV + t * _K1_CMB, _K1_CMB)])

    padv = jnp.full((LANES,), NP, jnp.int32)
    for k in range(_K1_CHUNK // LANES):
        idx2[_K1_FULL, pl.ds(k * LANES, LANES)] = padv

    base = c * E + s * _K1_EPT

    def load(r, _):
        pltpu.sync_copy(edges_hbm.at[pl.ds(base + r * _K1_CHUNK, _K1_CHUNK)],
                        idx2.at[r])
        return 0

    lax.fori_loop(0, _K1_FULL, load, 0)
    pltpu.sync_copy(edges_hbm.at[pl.ds(base + _K1_FULL * _K1_CHUNK, _K1_TAIL)],
                    idx2.at[_K1_FULL, pl.ds(0, _K1_TAIL)])

    shift = s * _K1_PRIV

    def shift_row(r, _):
        for g in range(_K1_CHUNK // LANES):
            idx2[r, pl.ds(g * LANES, LANES)] = (
                idx2[r, pl.ds(g * LANES, LANES)] + shift)
        return 0

    lax.fori_loop(0, _K1_FULL + 1, shift_row, 0)

    def hist(r, _):
        pltpu.sync_copy(ones_r, parts.at[idx2.at[r]], add=True)
        return 0

    lax.fori_loop(0, _K1_FULL + 1, hist, 0)

    plsc.subcore_barrier()

    for t in range(NS):
        pltpu.sync_copy(parts.at[pl.ds(t * _K1_PRIV + s * _K1_CMB, _K1_CMB)],
                        cbuf.at[pl.ds(t * _K1_CMB, _K1_CMB)])

    def cmb(g, _):
        v = cbuf[pl.ds(g * LANES, LANES)]
        for t in range(1, NS):
            v = v + cbuf[pl.ds(t * _K1_CMB + g * LANES, LANES)]
        dbuf[pl.ds(g * LANES, LANES)] = v
        return 0

    lax.fori_loop(0, _K1_CMB // LANES, cmb, 0)

    @pl.when(s < NS - 1)
    def _():
        pltpu.sync_copy(dbuf, degs_hbm.at[pl.ds(c * NP + s * _K1_CMB,
                                                _K1_CMB)])

    @pl.when(s == NS - 1)
    def _():
        pltpu.sync_copy(
            dbuf.at[pl.ds(0, _K1_LAST)],
            degs_hbm.at[pl.ds(c * NP + (NS - 1) * _K1_CMB, _K1_LAST)])


def _degrees(edges_flat):
    return pl.kernel(
        _deg_body,
        out_type=jax.ShapeDtypeStruct((2 * NP,), jnp.float32),
        mesh=_mesh(),
        scratch_types=[
            pltpu.VMEM((_K1_FULL + 1, _K1_CHUNK), jnp.int32),
            pltpu.VMEM((_K1_CHUNK,), jnp.float32),
            pltpu.VMEM((_K1_CMB,), jnp.float32),
            pltpu.VMEM((_K1_PRIV,), jnp.float32),
            pltpu.VMEM_SHARED((NS * _K1_PRIV,), jnp.float32),
        ],
        compiler_params=_SC_PARAMS,
    )(edges_flat)


_AGG_B = 80
_AGG_EPT = E // (NC * NS)
_AGG_STEPS = _AGG_EPT // _AGG_B
_AGG_RPT = NP // NS


def _agg_body(table_hbm, src_hbm, dst_hbm, part_hbm, sidx, didx, rows, acc):
    c = lax.axis_index("c")
    s = lax.axis_index("s")
    wid = s * NC + c

    zv = jnp.zeros((LANES,), jnp.float32)

    def zr(r, _):
        for k in range(H // LANES):
            rows[r, pl.ds(k * LANES, LANES)] = zv
        return 0

    lax.fori_loop(0, _AGG_B, zr, 0)
    for k in range(_AGG_RPT // _AGG_B):
        pltpu.sync_copy(rows, acc.at[pl.ds(s * _AGG_RPT + k * _AGG_B, _AGG_B)])
    plsc.subcore_barrier()

    ebase = wid * _AGG_EPT

    def step(i, _):
        base = ebase + i * _AGG_B
        pltpu.sync_copy(src_hbm.at[pl.ds(base, _AGG_B)], sidx)
        pltpu.sync_copy(dst_hbm.at[pl.ds(base, _AGG_B)], didx.at[0])
        pltpu.sync_copy(table_hbm.at[sidx], rows)
        pltpu.sync_copy(rows, acc.at[didx.at[0]], add=True)
        return 0

    lax.fori_loop(0, _AGG_STEPS, step, 0)
    plsc.subcore_barrier()

    for k in range(_AGG_RPT // _AGG_B):
        r0 = s * _AGG_RPT + k * _AGG_B
        pltpu.sync_copy(acc.at[pl.ds(r0, _AGG_B)], rows)
        pltpu.sync_copy(rows, part_hbm.at[c, pl.ds(r0, _AGG_B)])


def _agg(table, src, dst):
    return pl.kernel(
        _agg_body,
        out_type=jax.ShapeDtypeStruct((NC, NP, H), jnp.float32),
        mesh=_mesh(),
        scratch_types=[
            pltpu.VMEM((_AGG_B,), jnp.int32),
            pltpu.VMEM((1, _AGG_B), jnp.int32),
            pltpu.VMEM((_AGG_B, H), jnp.float32),
            pltpu.VMEM_SHARED((NP, H), jnp.float32),
        ],
        compiler_params=_SC_PARAMS,
    )(table, src, dst)


_TB = 1024


def _norm_from_deg(d):
    return jnp.where(d > 0, lax.rsqrt(jnp.maximum(d, 1e-12)), 0.0)


def _mm1_body(x_ref, w_ref, dego_ref, degi_ref, o_ref, no_ref, ni_ref):
    no = _norm_from_deg(dego_ref[...])
    ni = _norm_from_deg(degi_ref[...])
    no_ref[...] = no
    ni_ref[...] = ni
    o_ref[...] = (
        jnp.dot(x_ref[...], w_ref[...], preferred_element_type=jnp.float32)
        * no
    )


def _mm1(xp, W1, dego2d, degi2d):
    return pl.pallas_call(
        _mm1_body,
        grid=(NP // _TB,),
        in_specs=[
            pl.BlockSpec((_TB, D), lambda i: (i, 0)),
            pl.BlockSpec((D, H), lambda i: (0, 0)),
            pl.BlockSpec((_TB, 1), lambda i: (i, 0)),
            pl.BlockSpec((_TB, 1), lambda i: (i, 0)),
        ],
        out_specs=[
            pl.BlockSpec((_TB, H), lambda i: (i, 0)),
            pl.BlockSpec((_TB, 1), lambda i: (i, 0)),
            pl.BlockSpec((_TB, 1), lambda i: (i, 0)),
        ],
        out_shape=[
            jax.ShapeDtypeStruct((NP, H), jnp.float32),
            jax.ShapeDtypeStruct((NP, 1), jnp.float32),
            jax.ShapeDtypeStruct((NP, 1), jnp.float32),
        ],
    )(xp, W1, dego2d, degi2d)


def _mid_body(p_ref, ni_ref, b1_ref, w2_ref, no_ref, o_ref):
    sacc = p_ref[0] + p_ref[1]
    h = jnp.maximum(sacc * ni_ref[...] + b1_ref[...], 0.0)
    o_ref[...] = (
        jnp.dot(h, w2_ref[...], preferred_element_type=jnp.float32)
        * no_ref[...]
    )


def _mid(p, ni2d, b1r, W2p, no2d):
    return pl.pallas_call(
        _mid_body,
        grid=(NP // _TB,),
        in_specs=[
            pl.BlockSpec((NC, _TB, H), lambda i: (0, i, 0)),
            pl.BlockSpec((_TB, 1), lambda i: (i, 0)),
            pl.BlockSpec((1, H), lambda i: (0, 0)),
            pl.BlockSpec((H, H), lambda i: (0, 0)),
            pl.BlockSpec((_TB, 1), lambda i: (i, 0)),
        ],
        out_specs=pl.BlockSpec((_TB, H), lambda i: (i, 0)),
        out_shape=jax.ShapeDtypeStruct((NP, H), jnp.float32),
    )(p, ni2d, b1r, W2p, no2d)


def _fin_body(q_ref, ni_ref, b2_ref, o_ref):
    o_ref[...] = (q_ref[0] + q_ref[1]) * ni_ref[...] + b2_ref[...]


def _fin(q, ni2d, b2r):
    return pl.pallas_call(
        _fin_body,
        grid=(NP // _TB,),
        in_specs=[
            pl.BlockSpec((NC, _TB, H), lambda i: (0, i, 0)),
            pl.BlockSpec((_TB, 1), lambda i: (i, 0)),
            pl.BlockSpec((1, H), lambda i: (0, 0)),
        ],
        out_specs=pl.BlockSpec((_TB, H), lambda i: (i, 0)),
        out_shape=jax.ShapeDtypeStruct((NP, H), jnp.float32),
    )(q, ni2d, b2r)


def kernel(in_feat, edge_index, W1, b1, W2, b2):
    src = edge_index[0]
    dst = edge_index[1]
    xp = jnp.zeros((NP, D), in_feat.dtype).at[:N].set(in_feat)
    W2p = jnp.zeros((H, H), W2.dtype).at[:, :C].set(W2)
    b2p = jnp.zeros((1, H), b2.dtype).at[0, :C].set(b2)

    degs = _degrees(edge_index.reshape(2 * E))
    xw, no2d, ni2d = _mm1(xp, W1, degs[:NP].reshape(NP, 1),
                          degs[NP:].reshape(NP, 1))
    p = _agg(xw, src, dst)
    t2 = _mid(p, ni2d, b1.reshape(1, H), W2p, no2d)
    q = _agg(t2, src, dst)
    out = _fin(q, ni2d, b2p)
    return out[:N, :C]


# speedup vs baseline: 8.7231x; 1.7018x over previous
import functools

import jax
import jax.numpy as jnp
from jax import lax
from jax.experimental import pallas as pl
from jax.experimental.pallas import tpu as pltpu
from jax.experimental.pallas import tpu_sc as plsc

N = 10000
NP = 10240
E = 320000
D = 128
H = 128
C = 16

NC = 2
NS = 16
LANES = 16

_SC_PARAMS = pltpu.CompilerParams(use_tc_tiling_on_sc=False)


def _mesh():
    return plsc.VectorSubcoreMesh(core_axis_name="c", subcore_axis_name="s")


_K1_CHUNK = 128
_K1_EPT = E // NS
_K1_FULL = _K1_EPT // _K1_CHUNK
_K1_TAIL = _K1_EPT - _K1_FULL * _K1_CHUNK
_K1_CMB = 656
_K1_PRIV = _K1_CMB * NS
_K1_LAST = NP - (NS - 1) * _K1_CMB


def _deg_body(edges_hbm, degs_hbm, idx2, ones_r, dbuf, cbuf, parts):
    c = lax.axis_index("c")
    s = lax.axis_index("s")

    zv = jnp.zeros((LANES,), jnp.float32)
    ov = jnp.ones((LANES,), jnp.float32)

    def fo(r, _):
        ones_r[pl.ds(r * LANES, LANES)] = ov
        return 0

    lax.fori_loop(0, _K1_CHUNK // LANES, fo, 0)

    def fz(r, _):
        dbuf[pl.ds(r * LANES, LANES)] = zv
        return 0

    lax.fori_loop(0, _K1_CMB // LANES, fz, 0)

    for t in range(NS):
        pltpu.sync_copy(dbuf,
                        parts.at[pl.ds(s * _K1_PRIV + t * _K1_CMB, _K1_CMB)])

    padv = jnp.full((LANES,), NP, jnp.int32)
    for k in range(_K1_CHUNK // LANES):
        idx2[_K1_FULL, pl.ds(k * LANES, LANES)] = padv

    base = c * E + s * _K1_EPT

    def load(r, _):
        pltpu.sync_copy(edges_hbm.at[pl.ds(base + r * _K1_CHUNK, _K1_CHUNK)],
                        idx2.at[r])
        return 0

    lax.fori_loop(0, _K1_FULL, load, 0)
    pltpu.sync_copy(edges_hbm.at[pl.ds(base + _K1_FULL * _K1_CHUNK, _K1_TAIL)],
                    idx2.at[_K1_FULL, pl.ds(0, _K1_TAIL)])

    shift = s * _K1_PRIV

    def shift_row(r, _):
        for g in range(_K1_CHUNK // LANES):
            idx2[r, pl.ds(g * LANES, LANES)] = (
                idx2[r, pl.ds(g * LANES, LANES)] + shift)
        return 0

    lax.fori_loop(0, _K1_FULL + 1, shift_row, 0)

    def hist(r, _):
        pltpu.sync_copy(ones_r, parts.at[idx2.at[r]], add=True)
        return 0

    lax.fori_loop(0, _K1_FULL + 1, hist, 0)

    plsc.subcore_barrier()

    for t in range(NS):
        pltpu.sync_copy(parts.at[pl.ds(t * _K1_PRIV + s * _K1_CMB, _K1_CMB)],
                        cbuf.at[pl.ds(t * _K1_CMB, _K1_CMB)])

    def cmb(g, _):
        v = cbuf[pl.ds(g * LANES, LANES)]
        for t in range(1, NS):
            v = v + cbuf[pl.ds(t * _K1_CMB + g * LANES, LANES)]
        dbuf[pl.ds(g * LANES, LANES)] = v
        return 0

    lax.fori_loop(0, _K1_CMB // LANES, cmb, 0)

    @pl.when(s < NS - 1)
    def _():
        pltpu.sync_copy(dbuf, degs_hbm.at[pl.ds(c * NP + s * _K1_CMB,
                                                _K1_CMB)])

    @pl.when(s == NS - 1)
    def _():
        pltpu.sync_copy(
            dbuf.at[pl.ds(0, _K1_LAST)],
            degs_hbm.at[pl.ds(c * NP + (NS - 1) * _K1_CMB, _K1_LAST)])


def _degrees(edges_flat):
    return pl.kernel(
        _deg_body,
        out_type=jax.ShapeDtypeStruct((2 * NP,), jnp.float32),
        mesh=_mesh(),
        scratch_types=[
            pltpu.VMEM((_K1_FULL + 1, _K1_CHUNK), jnp.int32),
            pltpu.VMEM((_K1_CHUNK,), jnp.float32),
            pltpu.VMEM((_K1_CMB,), jnp.float32),
            pltpu.VMEM((_K1_PRIV,), jnp.float32),
            pltpu.VMEM_SHARED((NS * _K1_PRIV,), jnp.float32),
        ],
        compiler_params=_SC_PARAMS,
    )(edges_flat)


_AGG_B = 128
_EROWS = E // _AGG_B
_AGG_RT = _EROWS // (NC * NS)
_AGG_XTRA = _EROWS - _AGG_RT * NC * NS
_AGG_RPT = NP // NS
_AGG_OCH = 64


def _agg_body(table_hbm, src_hbm, dst_hbm, part_hbm, sidx_a, sidx_b,
              didx_a, didx_b, rows_a, rows_b, sem_a, sem_b, acc):
    c = lax.axis_index("c")
    s = lax.axis_index("s")
    wid = s * NC + c

    zv = jnp.zeros((LANES,), jnp.float32)

    def zr(r, _):
        for k in range(H // LANES):
            rows_a[r, pl.ds(k * LANES, LANES)] = zv
        return 0

    lax.fori_loop(0, _AGG_OCH, zr, 0)
    for k in range(_AGG_RPT // _AGG_OCH):
        pltpu.sync_copy(rows_a.at[pl.ds(0, _AGG_OCH)],
                        acc.at[pl.ds(s * _AGG_RPT + k * _AGG_OCH, _AGG_OCH)])
    plsc.subcore_barrier()

    e0 = wid * _AGG_RT * _AGG_B

    def eoff(j):
        return e0 + j * _AGG_B

    pltpu.sync_copy(src_hbm.at[pl.ds(eoff(0), _AGG_B)], sidx_a)
    pltpu.sync_copy(dst_hbm.at[pl.ds(eoff(0), _AGG_B)], didx_a.at[0])
    pltpu.async_copy(table_hbm.at[sidx_a], rows_a, sem_a)

    def pair(k, _):
        j0 = 2 * k
        j1 = j0 + 1
        pltpu.sync_copy(src_hbm.at[pl.ds(eoff(j1), _AGG_B)], sidx_b)
        pltpu.make_async_copy(table_hbm.at[sidx_a], rows_a, sem_a).wait()
        pltpu.async_copy(table_hbm.at[sidx_b], rows_b, sem_b)
        pltpu.sync_copy(dst_hbm.at[pl.ds(eoff(j1), _AGG_B)], didx_b.at[0])
        pltpu.sync_copy(rows_a, acc.at[didx_a.at[0]], add=True)

        @pl.when(j0 + 2 < _AGG_RT)
        def _():
            pltpu.sync_copy(src_hbm.at[pl.ds(eoff(j0 + 2), _AGG_B)], sidx_a)

        pltpu.make_async_copy(table_hbm.at[sidx_b], rows_b, sem_b).wait()

        @pl.when(j0 + 2 < _AGG_RT)
        def _():
            pltpu.async_copy(table_hbm.at[sidx_a], rows_a, sem_a)
            pltpu.sync_copy(dst_hbm.at[pl.ds(eoff(j0 + 2), _AGG_B)],
                            didx_a.at[0])

        pltpu.sync_copy(rows_b, acc.at[didx_b.at[0]], add=True)
        return 0

    lax.fori_loop(0, _AGG_RT // 2, pair, 0)

    @pl.when(wid < _AGG_XTRA)
    def _():
        xe = (_EROWS - _AGG_XTRA + wid) * _AGG_B
        pltpu.sync_copy(src_hbm.at[pl.ds(xe, _AGG_B)], sidx_a)
        pltpu.sync_copy(dst_hbm.at[pl.ds(xe, _AGG_B)], didx_a.at[0])
        pltpu.sync_copy(table_hbm.at[sidx_a], rows_a)
        pltpu.sync_copy(rows_a, acc.at[didx_a.at[0]], add=True)

    plsc.subcore_barrier()

    for k in range(_AGG_RPT // _AGG_OCH):
        rr = s * _AGG_RPT + k * _AGG_OCH
        pltpu.sync_copy(acc.at[pl.ds(rr, _AGG_OCH)],
                        rows_a.at[pl.ds(0, _AGG_OCH)])
        pltpu.sync_copy(rows_a.at[pl.ds(0, _AGG_OCH)],
                        part_hbm.at[c, pl.ds(rr, _AGG_OCH)])


def _agg(table, src, dst):
    return pl.kernel(
        _agg_body,
        out_type=jax.ShapeDtypeStruct((NC, NP, H), jnp.float32),
        mesh=_mesh(),
        scratch_types=[
            pltpu.VMEM((_AGG_B,), jnp.int32),
            pltpu.VMEM((_AGG_B,), jnp.int32),
            pltpu.VMEM((1, _AGG_B), jnp.int32),
            pltpu.VMEM((1, _AGG_B), jnp.int32),
            pltpu.VMEM((_AGG_B, H), jnp.float32),
            pltpu.VMEM((_AGG_B, H), jnp.float32),
            pltpu.SemaphoreType.DMA,
            pltpu.SemaphoreType.DMA,
            pltpu.VMEM_SHARED((NP, H), jnp.float32),
        ],
        compiler_params=_SC_PARAMS,
    )(table, src, dst)


_TB = 1024


def _norm_from_deg(d):
    return jnp.where(d > 0, lax.rsqrt(jnp.maximum(d, 1e-12)), 0.0)


def _mm1_body(x_ref, w_ref, dego_ref, degi_ref, o_ref, no_ref, ni_ref):
    no = _norm_from_deg(dego_ref[...])
    ni = _norm_from_deg(degi_ref[...])
    no_ref[...] = no
    ni_ref[...] = ni
    o_ref[...] = (
        jnp.dot(x_ref[...], w_ref[...], preferred_element_type=jnp.float32)
        * no
    )


def _mm1(xp, W1, dego2d, degi2d):
    return pl.pallas_call(
        _mm1_body,
        grid=(NP // _TB,),
        in_specs=[
            pl.BlockSpec((_TB, D), lambda i: (i, 0)),
            pl.BlockSpec((D, H), lambda i: (0, 0)),
            pl.BlockSpec((_TB, 1), lambda i: (i, 0)),
            pl.BlockSpec((_TB, 1), lambda i: (i, 0)),
        ],
        out_specs=[
            pl.BlockSpec((_TB, H), lambda i: (i, 0)),
            pl.BlockSpec((_TB, 1), lambda i: (i, 0)),
            pl.BlockSpec((_TB, 1), lambda i: (i, 0)),
        ],
        out_shape=[
            jax.ShapeDtypeStruct((NP, H), jnp.float32),
            jax.ShapeDtypeStruct((NP, 1), jnp.float32),
            jax.ShapeDtypeStruct((NP, 1), jnp.float32),
        ],
    )(xp, W1, dego2d, degi2d)


def _mid_body(p_ref, ni_ref, b1_ref, w2_ref, no_ref, o_ref):
    sacc = p_ref[0] + p_ref[1]
    h = jnp.maximum(sacc * ni_ref[...] + b1_ref[...], 0.0)
    o_ref[...] = (
        jnp.dot(h, w2_ref[...], preferred_element_type=jnp.float32)
        * no_ref[...]
    )


def _mid(p, ni2d, b1r, W2p, no2d):
    return pl.pallas_call(
        _mid_body,
        grid=(NP // _TB,),
        in_specs=[
            pl.BlockSpec((NC, _TB, H), lambda i: (0, i, 0)),
            pl.BlockSpec((_TB, 1), lambda i: (i, 0)),
            pl.BlockSpec((1, H), lambda i: (0, 0)),
            pl.BlockSpec((H, H), lambda i: (0, 0)),
            pl.BlockSpec((_TB, 1), lambda i: (i, 0)),
        ],
        out_specs=pl.BlockSpec((_TB, H), lambda i: (i, 0)),
        out_shape=jax.ShapeDtypeStruct((NP, H), jnp.float32),
    )(p, ni2d, b1r, W2p, no2d)


def _fin_body(q_ref, ni_ref, b2_ref, o_ref):
    o_ref[...] = (q_ref[0] + q_ref[1]) * ni_ref[...] + b2_ref[...]


def _fin(q, ni2d, b2r):
    return pl.pallas_call(
        _fin_body,
        grid=(NP // _TB,),
        in_specs=[
            pl.BlockSpec((NC, _TB, H), lambda i: (0, i, 0)),
            pl.BlockSpec((_TB, 1), lambda i: (i, 0)),
            pl.BlockSpec((1, H), lambda i: (0, 0)),
        ],
        out_specs=pl.BlockSpec((_TB, H), lambda i: (i, 0)),
        out_shape=jax.ShapeDtypeStruct((NP, H), jnp.float32),
    )(q, ni2d, b2r)


def kernel(in_feat, edge_index, W1, b1, W2, b2):
    src = edge_index[0]
    dst = edge_index[1]
    xp = jnp.zeros((NP, D), in_feat.dtype).at[:N].set(in_feat)
    W2p = jnp.zeros((H, H), W2.dtype).at[:, :C].set(W2)
    b2p = jnp.zeros((1, H), b2.dtype).at[0, :C].set(b2)

    degs = _degrees(edge_index.reshape(2 * E))
    xw, no2d, ni2d = _mm1(xp, W1, degs[:NP].reshape(NP, 1),
                          degs[NP:].reshape(NP, 1))
    p = _agg(xw, src, dst)
    t2 = _mid(p, ni2d, b1.reshape(1, H), W2p, no2d)
    q = _agg(t2, src, dst)
    out = _fin(q, ni2d, b2p)
    return out[:N, :C]


# speedup vs baseline: 10.5312x; 1.2073x over previous
import functools

import jax
import jax.numpy as jnp
from jax import lax
from jax.experimental import pallas as pl
from jax.experimental.pallas import tpu as pltpu
from jax.experimental.pallas import tpu_sc as plsc

N = 10000
NP = 10240
E = 320000
D = 128
H = 128
C = 16

NC = 2
NS = 16
LANES = 16

_SC_PARAMS = pltpu.CompilerParams(use_tc_tiling_on_sc=False)


def _mesh():
    return plsc.VectorSubcoreMesh(core_axis_name="c", subcore_axis_name="s")


_K1_CHUNK = 128
_K1_ROWS = E // _K1_CHUNK
_K1_RT = _K1_ROWS // NS
_K1_XTRA = _K1_ROWS - _K1_RT * NS
_K1_CMB = 656
_K1_PRIV = _K1_CMB * NS
_K1_LAST = NP - (NS - 1) * _K1_CMB
_K1_DEPTH = 8


def _deg_body(edges_hbm, degs_hbm, idx2, ones_r, dbuf, cbuf, semh, parts):
    c = lax.axis_index("c")
    s = lax.axis_index("s")

    zv = jnp.zeros((LANES,), jnp.float32)
    ov = jnp.ones((LANES,), jnp.float32)

    def fo(r, _):
        ones_r[pl.ds(r * LANES, LANES)] = ov
        return 0

    lax.fori_loop(0, _K1_CHUNK // LANES, fo, 0)

    def fz(r, _):
        dbuf[pl.ds(r * LANES, LANES)] = zv
        return 0

    lax.fori_loop(0, _K1_CMB // LANES, fz, 0)

    for t in range(NS):
        pltpu.async_copy(
            dbuf, parts.at[pl.ds(s * _K1_PRIV + t * _K1_CMB, _K1_CMB)], semh)
    for t in range(NS):
        pltpu.make_async_copy(
            dbuf, parts.at[pl.ds(s * _K1_PRIV + t * _K1_CMB, _K1_CMB)],
            semh).wait()

    r0 = c * _K1_ROWS + s * _K1_RT
    pltpu.sync_copy(edges_hbm.at[pl.ds(r0, _K1_RT)], idx2.at[pl.ds(0, _K1_RT)])

    @pl.when(s < _K1_XTRA)
    def _():
        xr = c * _K1_ROWS + _K1_ROWS - _K1_XTRA + s
        pltpu.sync_copy(edges_hbm.at[xr], idx2.at[_K1_RT])

    shift = s * _K1_PRIV

    def shift_row(r, _):
        for g in range(_K1_CHUNK // LANES):
            idx2[r, pl.ds(g * LANES, LANES)] = (
                idx2[r, pl.ds(g * LANES, LANES)] + shift)
        return 0

    lax.fori_loop(0, _K1_RT + 1, shift_row, 0)

    nrows = jnp.where(s < _K1_XTRA, _K1_RT + 1, _K1_RT)

    def fire(r):
        pltpu.async_copy(ones_r, parts.at[idx2.at[r]], semh, add=True)

    for r in range(_K1_DEPTH):
        fire(r)

    def ring(r, _):
        pltpu.make_async_copy(ones_r, parts.at[idx2.at[r]], semh).wait()

        @pl.when(r + _K1_DEPTH < nrows)
        def _():
            fire(r + _K1_DEPTH)

        return 0

    lax.fori_loop(0, nrows, ring, 0)

    plsc.subcore_barrier()

    for t in range(NS):
        pltpu.async_copy(
            parts.at[pl.ds(t * _K1_PRIV + s * _K1_CMB, _K1_CMB)],
            cbuf.at[pl.ds(t * _K1_CMB, _K1_CMB)], semh)
    for t in range(NS):
        pltpu.make_async_copy(
            parts.at[pl.ds(t * _K1_PRIV + s * _K1_CMB, _K1_CMB)],
            cbuf.at[pl.ds(t * _K1_CMB, _K1_CMB)], semh).wait()

    def cmb(g, _):
        v = cbuf[pl.ds(g * LANES, LANES)]
        for t in range(1, NS):
            v = v + cbuf[pl.ds(t * _K1_CMB + g * LANES, LANES)]
        dbuf[pl.ds(g * LANES, LANES)] = v
        return 0

    lax.fori_loop(0, _K1_CMB // LANES, cmb, 0)

    @pl.when(s < NS - 1)
    def _():
        pltpu.sync_copy(dbuf, degs_hbm.at[pl.ds(c * NP + s * _K1_CMB,
                                                _K1_CMB)])

    @pl.when(s == NS - 1)
    def _():
        pltpu.sync_copy(
            dbuf.at[pl.ds(0, _K1_LAST)],
            degs_hbm.at[pl.ds(c * NP + (NS - 1) * _K1_CMB, _K1_LAST)])


def _degrees(edges2d):
    return pl.kernel(
        _deg_body,
        out_type=jax.ShapeDtypeStruct((2 * NP,), jnp.float32),
        mesh=_mesh(),
        scratch_types=[
            pltpu.VMEM((_K1_RT + 1, _K1_CHUNK), jnp.int32),
            pltpu.VMEM((_K1_CHUNK,), jnp.float32),
            pltpu.VMEM((_K1_CMB,), jnp.float32),
            pltpu.VMEM((_K1_PRIV,), jnp.float32),
            pltpu.SemaphoreType.DMA,
            pltpu.VMEM_SHARED((NS * _K1_PRIV,), jnp.float32),
        ],
        compiler_params=_SC_PARAMS,
    )(edges2d)


_AGG_B = 128
_EROWS = E // _AGG_B
_AGG_RT = _EROWS // (NC * NS)
_AGG_XTRA = _EROWS - _AGG_RT * NC * NS
_AGG_RPT = NP // NS
_AGG_OCH = 64


def _agg_body(table_hbm, src_hbm, dst_hbm, part_hbm, sidx_a, sidx_b,
              didx_a, didx_b, rows_a, rows_b, sem_a, sem_b, acc):
    c = lax.axis_index("c")
    s = lax.axis_index("s")
    wid = s * NC + c

    zv = jnp.zeros((LANES,), jnp.float32)

    def zr(r, _):
        for k in range(H // LANES):
            rows_a[r, pl.ds(k * LANES, LANES)] = zv
        return 0

    lax.fori_loop(0, _AGG_OCH, zr, 0)
    for k in range(_AGG_RPT // _AGG_OCH):
        pltpu.sync_copy(rows_a.at[pl.ds(0, _AGG_OCH)],
                        acc.at[pl.ds(s * _AGG_RPT + k * _AGG_OCH, _AGG_OCH)])
    plsc.subcore_barrier()

    e0 = wid * _AGG_RT * _AGG_B

    def eoff(j):
        return e0 + j * _AGG_B

    pltpu.sync_copy(src_hbm.at[pl.ds(eoff(0), _AGG_B)], sidx_a)
    pltpu.sync_copy(dst_hbm.at[pl.ds(eoff(0), _AGG_B)], didx_a.at[0])
    pltpu.async_copy(table_hbm.at[sidx_a], rows_a, sem_a)

    def pair(k, _):
        j0 = 2 * k
        j1 = j0 + 1
        pltpu.sync_copy(src_hbm.at[pl.ds(eoff(j1), _AGG_B)], sidx_b)
        pltpu.make_async_copy(table_hbm.at[sidx_a], rows_a, sem_a).wait()
        pltpu.async_copy(table_hbm.at[sidx_b], rows_b, sem_b)
        pltpu.sync_copy(dst_hbm.at[pl.ds(eoff(j1), _AGG_B)], didx_b.at[0])
        pltpu.sync_copy(rows_a, acc.at[didx_a.at[0]], add=True)

        @pl.when(j0 + 2 < _AGG_RT)
        def _():
            pltpu.sync_copy(src_hbm.at[pl.ds(eoff(j0 + 2), _AGG_B)], sidx_a)

        pltpu.make_async_copy(table_hbm.at[sidx_b], rows_b, sem_b).wait()

        @pl.when(j0 + 2 < _AGG_RT)
        def _():
            pltpu.async_copy(table_hbm.at[sidx_a], rows_a, sem_a)
            pltpu.sync_copy(dst_hbm.at[pl.ds(eoff(j0 + 2), _AGG_B)],
                            didx_a.at[0])

        pltpu.sync_copy(rows_b, acc.at[didx_b.at[0]], add=True)
        return 0

    lax.fori_loop(0, _AGG_RT // 2, pair, 0)

    @pl.when(wid < _AGG_XTRA)
    def _():
        xe = (_EROWS - _AGG_XTRA + wid) * _AGG_B
        pltpu.sync_copy(src_hbm.at[pl.ds(xe, _AGG_B)], sidx_a)
        pltpu.sync_copy(dst_hbm.at[pl.ds(xe, _AGG_B)], didx_a.at[0])
        pltpu.sync_copy(table_hbm.at[sidx_a], rows_a)
        pltpu.sync_copy(rows_a, acc.at[didx_a.at[0]], add=True)

    plsc.subcore_barrier()

    for k in range(_AGG_RPT // _AGG_OCH):
        rr = s * _AGG_RPT + k * _AGG_OCH
        pltpu.sync_copy(acc.at[pl.ds(rr, _AGG_OCH)],
                        rows_a.at[pl.ds(0, _AGG_OCH)])
        pltpu.sync_copy(rows_a.at[pl.ds(0, _AGG_OCH)],
                        part_hbm.at[c, pl.ds(rr, _AGG_OCH)])


def _agg(table, src, dst):
    return pl.kernel(
        _agg_body,
        out_type=jax.ShapeDtypeStruct((NC, NP, H), jnp.float32),
        mesh=_mesh(),
        scratch_types=[
            pltpu.VMEM((_AGG_B,), jnp.int32),
            pltpu.VMEM((_AGG_B,), jnp.int32),
            pltpu.VMEM((1, _AGG_B), jnp.int32),
            pltpu.VMEM((1, _AGG_B), jnp.int32),
            pltpu.VMEM((_AGG_B, H), jnp.float32),
            pltpu.VMEM((_AGG_B, H), jnp.float32),
            pltpu.SemaphoreType.DMA,
            pltpu.SemaphoreType.DMA,
            pltpu.VMEM_SHARED((NP, H), jnp.float32),
        ],
        compiler_params=_SC_PARAMS,
    )(table, src, dst)


_TB = 1024


def _norm_from_deg(d):
    return jnp.where(d > 0, lax.rsqrt(jnp.maximum(d, 1e-12)), 0.0)


def _mm1_body(x_ref, w_ref, dego_ref, degi_ref, o_ref, no_ref, ni_ref):
    no = _norm_from_deg(dego_ref[...])
    ni = _norm_from_deg(degi_ref[...])
    no_ref[...] = no
    ni_ref[...] = ni
    o_ref[...] = (
        jnp.dot(x_ref[...], w_ref[...], preferred_element_type=jnp.float32)
        * no
    )


def _mm1(xp, W1, dego2d, degi2d):
    return pl.pallas_call(
        _mm1_body,
        grid=(NP // _TB,),
        in_specs=[
            pl.BlockSpec((_TB, D), lambda i: (i, 0)),
            pl.BlockSpec((D, H), lambda i: (0, 0)),
            pl.BlockSpec((_TB, 1), lambda i: (i, 0)),
            pl.BlockSpec((_TB, 1), lambda i: (i, 0)),
        ],
        out_specs=[
            pl.BlockSpec((_TB, H), lambda i: (i, 0)),
            pl.BlockSpec((_TB, 1), lambda i: (i, 0)),
            pl.BlockSpec((_TB, 1), lambda i: (i, 0)),
        ],
        out_shape=[
            jax.ShapeDtypeStruct((NP, H), jnp.float32),
            jax.ShapeDtypeStruct((NP, 1), jnp.float32),
            jax.ShapeDtypeStruct((NP, 1), jnp.float32),
        ],
    )(xp, W1, dego2d, degi2d)


def _mid_body(p_ref, ni_ref, b1_ref, w2_ref, no_ref, o_ref):
    sacc = p_ref[0] + p_ref[1]
    h = jnp.maximum(sacc * ni_ref[...] + b1_ref[...], 0.0)
    o_ref[...] = (
        jnp.dot(h, w2_ref[...], preferred_element_type=jnp.float32)
        * no_ref[...]
    )


def _mid(p, ni2d, b1r, W2p, no2d):
    return pl.pallas_call(
        _mid_body,
        grid=(NP // _TB,),
        in_specs=[
            pl.BlockSpec((NC, _TB, H), lambda i: (0, i, 0)),
            pl.BlockSpec((_TB, 1), lambda i: (i, 0)),
            pl.BlockSpec((1, H), lambda i: (0, 0)),
            pl.BlockSpec((H, H), lambda i: (0, 0)),
            pl.BlockSpec((_TB, 1), lambda i: (i, 0)),
        ],
        out_specs=pl.BlockSpec((_TB, H), lambda i: (i, 0)),
        out_shape=jax.ShapeDtypeStruct((NP, H), jnp.float32),
    )(p, ni2d, b1r, W2p, no2d)


def _fin_body(q_ref, ni_ref, b2_ref, o_ref):
    o_ref[...] = (q_ref[0] + q_ref[1]) * ni_ref[...] + b2_ref[...]


def _fin(q, ni2d, b2r):
    return pl.pallas_call(
        _fin_body,
        grid=(NP // _TB,),
        in_specs=[
            pl.BlockSpec((NC, _TB, H), lambda i: (0, i, 0)),
            pl.BlockSpec((_TB, 1), lambda i: (i, 0)),
            pl.BlockSpec((1, H), lambda i: (0, 0)),
        ],
        out_specs=pl.BlockSpec((_TB, H), lambda i: (i, 0)),
        out_shape=jax.ShapeDtypeStruct((NP, H), jnp.float32),
    )(q, ni2d, b2r)


def kernel(in_feat, edge_index, W1, b1, W2, b2):
    src = edge_index[0]
    dst = edge_index[1]
    xp = jnp.zeros((NP, D), in_feat.dtype).at[:N].set(in_feat)
    W2p = jnp.zeros((H, H), W2.dtype).at[:, :C].set(W2)
    b2p = jnp.zeros((1, H), b2.dtype).at[0, :C].set(b2)

    degs = _degrees(edge_index.reshape(2 * E // 128, 128))
    xw, no2d, ni2d = _mm1(xp, W1, degs[:NP].reshape(NP, 1),
                          degs[NP:].reshape(NP, 1))
    p = _agg(xw, src, dst)
    t2 = _mid(p, ni2d, b1.reshape(1, H), W2p, no2d)
    q = _agg(t2, src, dst)
    out = _fin(q, ni2d, b2p)
    return out[:N, :C]


# speedup vs baseline: 11.6359x; 1.1049x over previous
import functools

import jax
import jax.numpy as jnp
from jax import lax
from jax.experimental import pallas as pl
from jax.experimental.pallas import tpu as pltpu
from jax.experimental.pallas import tpu_sc as plsc

N = 10000
NP = 10240
E = 320000
D = 128
H = 128
C = 16

NC = 2
NS = 16
LANES = 16

_SC_PARAMS = pltpu.CompilerParams(use_tc_tiling_on_sc=False)


def _mesh():
    return plsc.VectorSubcoreMesh(core_axis_name="c", subcore_axis_name="s")


_K1_CHUNK = 128
_K1_ROWS = E // _K1_CHUNK
_K1_RT = _K1_ROWS // NS
_K1_XTRA = _K1_ROWS - _K1_RT * NS
_K1_CMB = 656
_K1_PRIV = _K1_CMB * NS
_K1_LAST = NP - (NS - 1) * _K1_CMB
_K1_DEPTH = 8


def _deg_body(edges_hbm, degs_hbm, idx2, ones_r, dbuf, cbuf, semh, parts):
    c = lax.axis_index("c")
    s = lax.axis_index("s")

    zv = jnp.zeros((LANES,), jnp.float32)
    ov = jnp.ones((LANES,), jnp.float32)

    def fo(r, _):
        ones_r[pl.ds(r * LANES, LANES)] = ov
        return 0

    lax.fori_loop(0, _K1_CHUNK // LANES, fo, 0)

    def fz(r, _):
        dbuf[pl.ds(r * LANES, LANES)] = zv
        return 0

    lax.fori_loop(0, _K1_CMB // LANES, fz, 0)

    for t in range(NS):
        pltpu.async_copy(
            dbuf, parts.at[pl.ds(s * _K1_PRIV + t * _K1_CMB, _K1_CMB)], semh)
    for t in range(NS):
        pltpu.make_async_copy(
            dbuf, parts.at[pl.ds(s * _K1_PRIV + t * _K1_CMB, _K1_CMB)],
            semh).wait()

    r0 = c * _K1_ROWS + s * _K1_RT
    pltpu.sync_copy(edges_hbm.at[pl.ds(r0, _K1_RT)], idx2.at[pl.ds(0, _K1_RT)])

    @pl.when(s < _K1_XTRA)
    def _():
        xr = c * _K1_ROWS + _K1_ROWS - _K1_XTRA + s
        pltpu.sync_copy(edges_hbm.at[xr], idx2.at[_K1_RT])

    shift = s * _K1_PRIV

    def shift_row(r, _):
        for g in range(_K1_CHUNK // LANES):
            idx2[r, pl.ds(g * LANES, LANES)] = (
                idx2[r, pl.ds(g * LANES, LANES)] + shift)
        return 0

    lax.fori_loop(0, _K1_RT + 1, shift_row, 0)

    nrows = jnp.where(s < _K1_XTRA, _K1_RT + 1, _K1_RT)

    def fire(r):
        pltpu.async_copy(ones_r, parts.at[idx2.at[r]], semh, add=True)

    for r in range(_K1_DEPTH):
        fire(r)

    def ring(r, _):
        pltpu.make_async_copy(ones_r, parts.at[idx2.at[r]], semh).wait()

        @pl.when(r + _K1_DEPTH < nrows)
        def _():
            fire(r + _K1_DEPTH)

        return 0

    lax.fori_loop(0, nrows, ring, 0)

    plsc.subcore_barrier()

    for t in range(NS):
        pltpu.async_copy(
            parts.at[pl.ds(t * _K1_PRIV + s * _K1_CMB, _K1_CMB)],
            cbuf.at[pl.ds(t * _K1_CMB, _K1_CMB)], semh)
    for t in range(NS):
        pltpu.make_async_copy(
            parts.at[pl.ds(t * _K1_PRIV + s * _K1_CMB, _K1_CMB)],
            cbuf.at[pl.ds(t * _K1_CMB, _K1_CMB)], semh).wait()

    def cmb(g, _):
        v = cbuf[pl.ds(g * LANES, LANES)]
        for t in range(1, NS):
            v = v + cbuf[pl.ds(t * _K1_CMB + g * LANES, LANES)]
        dbuf[pl.ds(g * LANES, LANES)] = v
        return 0

    lax.fori_loop(0, _K1_CMB // LANES, cmb, 0)

    @pl.when(s < NS - 1)
    def _():
        pltpu.sync_copy(dbuf, degs_hbm.at[pl.ds(c * NP + s * _K1_CMB,
                                                _K1_CMB)])

    @pl.when(s == NS - 1)
    def _():
        pltpu.sync_copy(
            dbuf.at[pl.ds(0, _K1_LAST)],
            degs_hbm.at[pl.ds(c * NP + (NS - 1) * _K1_CMB, _K1_LAST)])


def _degrees(edges2d):
    return pl.kernel(
        _deg_body,
        out_type=jax.ShapeDtypeStruct((2 * NP,), jnp.float32),
        mesh=_mesh(),
        scratch_types=[
            pltpu.VMEM((_K1_RT + 1, _K1_CHUNK), jnp.int32),
            pltpu.VMEM((_K1_CHUNK,), jnp.float32),
            pltpu.VMEM((_K1_CMB,), jnp.float32),
            pltpu.VMEM((_K1_PRIV,), jnp.float32),
            pltpu.SemaphoreType.DMA,
            pltpu.VMEM_SHARED((NS * _K1_PRIV,), jnp.float32),
        ],
        compiler_params=_SC_PARAMS,
    )(edges2d)


_AGG_B = 128
_EROWS = E // _AGG_B
_AGG_RT = _EROWS // (NC * NS)
_AGG_XTRA = _EROWS - _AGG_RT * NC * NS
_AGG_RPT = NP // NS
_AGG_OCH = 64


def _agg_body(table_hbm, src_hbm, dst_hbm, part_hbm, sidx_a, sidx_b,
              didx_a, didx_b, rows_a, rows_b, sem_a, sem_b, ssem_a, ssem_b,
              acc):
    c = lax.axis_index("c")
    s = lax.axis_index("s")
    wid = s * NC + c

    zv = jnp.zeros((LANES,), jnp.float32)

    def zr(r, _):
        for k in range(H // LANES):
            rows_a[r, pl.ds(k * LANES, LANES)] = zv
        return 0

    lax.fori_loop(0, _AGG_OCH, zr, 0)
    for k in range(_AGG_RPT // _AGG_OCH):
        pltpu.sync_copy(rows_a.at[pl.ds(0, _AGG_OCH)],
                        acc.at[pl.ds(s * _AGG_RPT + k * _AGG_OCH, _AGG_OCH)])
    plsc.subcore_barrier()

    e0 = wid * _AGG_RT * _AGG_B

    def eoff(j):
        return e0 + j * _AGG_B

    pltpu.sync_copy(src_hbm.at[pl.ds(eoff(0), _AGG_B)], sidx_a)
    pltpu.sync_copy(dst_hbm.at[pl.ds(eoff(0), _AGG_B)], didx_a.at[0])
    pltpu.async_copy(table_hbm.at[sidx_a], rows_a, sem_a)

    def pair(k, _):
        j0 = 2 * k
        j1 = j0 + 1
        pltpu.sync_copy(src_hbm.at[pl.ds(eoff(j1), _AGG_B)], sidx_b)
        pltpu.make_async_copy(table_hbm.at[sidx_a], rows_a, sem_a).wait()

        @pl.when(k > 0)
        def _():
            pltpu.make_async_copy(rows_b, acc.at[didx_b.at[0]], ssem_b).wait()

        pltpu.async_copy(table_hbm.at[sidx_b], rows_b, sem_b)
        pltpu.sync_copy(dst_hbm.at[pl.ds(eoff(j1), _AGG_B)], didx_b.at[0])
        pltpu.async_copy(rows_a, acc.at[didx_a.at[0]], ssem_a, add=True)

        @pl.when(j0 + 2 < _AGG_RT)
        def _():
            pltpu.sync_copy(src_hbm.at[pl.ds(eoff(j0 + 2), _AGG_B)], sidx_a)

        pltpu.make_async_copy(table_hbm.at[sidx_b], rows_b, sem_b).wait()
        pltpu.make_async_copy(rows_a, acc.at[didx_a.at[0]], ssem_a).wait()

        @pl.when(j0 + 2 < _AGG_RT)
        def _():
            pltpu.async_copy(table_hbm.at[sidx_a], rows_a, sem_a)
            pltpu.sync_copy(dst_hbm.at[pl.ds(eoff(j0 + 2), _AGG_B)],
                            didx_a.at[0])

        pltpu.async_copy(rows_b, acc.at[didx_b.at[0]], ssem_b, add=True)
        return 0

    lax.fori_loop(0, _AGG_RT // 2, pair, 0)
    pltpu.make_async_copy(rows_b, acc.at[didx_b.at[0]], ssem_b).wait()

    @pl.when(wid < _AGG_XTRA)
    def _():
        xe = (_EROWS - _AGG_XTRA + wid) * _AGG_B
        pltpu.sync_copy(src_hbm.at[pl.ds(xe, _AGG_B)], sidx_a)
        pltpu.sync_copy(dst_hbm.at[pl.ds(xe, _AGG_B)], didx_a.at[0])
        pltpu.sync_copy(table_hbm.at[sidx_a], rows_a)
        pltpu.sync_copy(rows_a, acc.at[didx_a.at[0]], add=True)

    plsc.subcore_barrier()

    for k in range(_AGG_RPT // _AGG_OCH):
        rr = s * _AGG_RPT + k * _AGG_OCH
        pltpu.sync_copy(acc.at[pl.ds(rr, _AGG_OCH)],
                        rows_a.at[pl.ds(0, _AGG_OCH)])
        pltpu.sync_copy(rows_a.at[pl.ds(0, _AGG_OCH)],
                        part_hbm.at[c, pl.ds(rr, _AGG_OCH)])


def _agg(table, src, dst):
    return pl.kernel(
        _agg_body,
        out_type=jax.ShapeDtypeStruct((NC, NP, H), jnp.float32),
        mesh=_mesh(),
        scratch_types=[
            pltpu.VMEM((_AGG_B,), jnp.int32),
            pltpu.VMEM((_AGG_B,), jnp.int32),
            pltpu.VMEM((1, _AGG_B), jnp.int32),
            pltpu.VMEM((1, _AGG_B), jnp.int32),
            pltpu.VMEM((_AGG_B, H), jnp.float32),
            pltpu.VMEM((_AGG_B, H), jnp.float32),
            pltpu.SemaphoreType.DMA,
            pltpu.SemaphoreType.DMA,
            pltpu.SemaphoreType.DMA,
            pltpu.SemaphoreType.DMA,
            pltpu.VMEM_SHARED((NP, H), jnp.float32),
        ],
        compiler_params=_SC_PARAMS,
    )(table, src, dst)


_TB = 1024


def _norm_from_deg(d):
    return jnp.where(d > 0, lax.rsqrt(jnp.maximum(d, 1e-12)), 0.0)


def _mm1_body(x_ref, w_ref, dego_ref, degi_ref, o_ref, no_ref, ni_ref):
    no = _norm_from_deg(dego_ref[...])
    ni = _norm_from_deg(degi_ref[...])
    no_ref[...] = no
    ni_ref[...] = ni
    o_ref[...] = (
        jnp.dot(x_ref[...], w_ref[...], preferred_element_type=jnp.float32)
        * no
    )


def _mm1(xp, W1, dego2d, degi2d):
    return pl.pallas_call(
        _mm1_body,
        grid=(NP // _TB,),
        in_specs=[
            pl.BlockSpec((_TB, D), lambda i: (i, 0)),
            pl.BlockSpec((D, H), lambda i: (0, 0)),
            pl.BlockSpec((_TB, 1), lambda i: (i, 0)),
            pl.BlockSpec((_TB, 1), lambda i: (i, 0)),
        ],
        out_specs=[
            pl.BlockSpec((_TB, H), lambda i: (i, 0)),
            pl.BlockSpec((_TB, 1), lambda i: (i, 0)),
            pl.BlockSpec((_TB, 1), lambda i: (i, 0)),
        ],
        out_shape=[
            jax.ShapeDtypeStruct((NP, H), jnp.float32),
            jax.ShapeDtypeStruct((NP, 1), jnp.float32),
            jax.ShapeDtypeStruct((NP, 1), jnp.float32),
        ],
    )(xp, W1, dego2d, degi2d)


def _mid_body(p_ref, ni_ref, b1_ref, w2_ref, no_ref, o_ref):
    sacc = p_ref[0] + p_ref[1]
    h = jnp.maximum(sacc * ni_ref[...] + b1_ref[...], 0.0)
    o_ref[...] = (
        jnp.dot(h, w2_ref[...], preferred_element_type=jnp.float32)
        * no_ref[...]
    )


def _mid(p, ni2d, b1r, W2p, no2d):
    return pl.pallas_call(
        _mid_body,
        grid=(NP // _TB,),
        in_specs=[
            pl.BlockSpec((NC, _TB, H), lambda i: (0, i, 0)),
            pl.BlockSpec((_TB, 1), lambda i: (i, 0)),
            pl.BlockSpec((1, H), lambda i: (0, 0)),
            pl.BlockSpec((H, H), lambda i: (0, 0)),
            pl.BlockSpec((_TB, 1), lambda i: (i, 0)),
        ],
        out_specs=pl.BlockSpec((_TB, H), lambda i: (i, 0)),
        out_shape=jax.ShapeDtypeStruct((NP, H), jnp.float32),
    )(p, ni2d, b1r, W2p, no2d)


def _fin_body(q_ref, ni_ref, b2_ref, o_ref):
    o_ref[...] = (q_ref[0] + q_ref[1]) * ni_ref[...] + b2_ref[...]


def _fin(q, ni2d, b2r):
    return pl.pallas_call(
        _fin_body,
        grid=(NP // _TB,),
        in_specs=[
            pl.BlockSpec((NC, _TB, H), lambda i: (0, i, 0)),
            pl.BlockSpec((_TB, 1), lambda i: (i, 0)),
            pl.BlockSpec((1, H), lambda i: (0, 0)),
        ],
        out_specs=pl.BlockSpec((_TB, H), lambda i: (i, 0)),
        out_shape=jax.ShapeDtypeStruct((NP, H), jnp.float32),
    )(q, ni2d, b2r)


def kernel(in_feat, edge_index, W1, b1, W2, b2):
    src = edge_index[0]
    dst = edge_index[1]
    xp = jnp.zeros((NP, D), in_feat.dtype).at[:N].set(in_feat)
    W2p = jnp.zeros((H, H), W2.dtype).at[:, :C].set(W2)
    b2p = jnp.zeros((1, H), b2.dtype).at[0, :C].set(b2)

    degs = _degrees(edge_index.reshape(2 * E // 128, 128))
    xw, no2d, ni2d = _mm1(xp, W1, degs[:NP].reshape(NP, 1),
                          degs[NP:].reshape(NP, 1))
    p = _agg(xw, src, dst)
    t2 = _mid(p, ni2d, b1.reshape(1, H), W2p, no2d)
    q = _agg(t2, src, dst)
    out = _fin(q, ni2d, b2p)
    return out[:N, :C]


# speedup vs baseline: 11.8252x; 1.0163x over previous
import functools

import jax
import jax.numpy as jnp
from jax import lax
from jax.experimental import pallas as pl
from jax.experimental.pallas import tpu as pltpu
from jax.experimental.pallas import tpu_sc as plsc

N = 10000
NP = 10240
E = 320000
D = 128
H = 128
C = 16

NC = 2
NS = 16
LANES = 16

_SC_PARAMS = pltpu.CompilerParams(use_tc_tiling_on_sc=False)


def _mesh():
    return plsc.VectorSubcoreMesh(core_axis_name="c", subcore_axis_name="s")


_K1_CHUNK = 128
_K1_ROWS = E // _K1_CHUNK
_K1_RT = _K1_ROWS // NS
_K1_XTRA = _K1_ROWS - _K1_RT * NS
_K1_CMB = 656
_K1_PRIV = _K1_CMB * NS
_K1_LAST = NP - (NS - 1) * _K1_CMB
_K1_DEPTH = 8


def _deg_body(edges_hbm, degs_hbm, idx2, ones_r, dbuf, cbuf, semh, parts):
    c = lax.axis_index("c")
    s = lax.axis_index("s")

    zv = jnp.zeros((LANES,), jnp.float32)
    ov = jnp.ones((LANES,), jnp.float32)

    def fo(r, _):
        ones_r[pl.ds(r * LANES, LANES)] = ov
        return 0

    lax.fori_loop(0, _K1_CHUNK // LANES, fo, 0)

    def fz(r, _):
        dbuf[pl.ds(r * LANES, LANES)] = zv
        return 0

    lax.fori_loop(0, _K1_CMB // LANES, fz, 0)

    for t in range(NS):
        pltpu.async_copy(
            dbuf, parts.at[pl.ds(s * _K1_PRIV + t * _K1_CMB, _K1_CMB)], semh)
    for t in range(NS):
        pltpu.make_async_copy(
            dbuf, parts.at[pl.ds(s * _K1_PRIV + t * _K1_CMB, _K1_CMB)],
            semh).wait()

    r0 = c * _K1_ROWS + s * _K1_RT
    pltpu.sync_copy(edges_hbm.at[pl.ds(r0, _K1_RT)], idx2.at[pl.ds(0, _K1_RT)])

    @pl.when(s < _K1_XTRA)
    def _():
        xr = c * _K1_ROWS + _K1_ROWS - _K1_XTRA + s
        pltpu.sync_copy(edges_hbm.at[xr], idx2.at[_K1_RT])

    shift = s * _K1_PRIV

    def shift_row(r, _):
        for g in range(_K1_CHUNK // LANES):
            idx2[r, pl.ds(g * LANES, LANES)] = (
                idx2[r, pl.ds(g * LANES, LANES)] + shift)
        return 0

    lax.fori_loop(0, _K1_RT + 1, shift_row, 0)

    nrows = jnp.where(s < _K1_XTRA, _K1_RT + 1, _K1_RT)

    def fire(r):
        pltpu.async_copy(ones_r, parts.at[idx2.at[r]], semh, add=True)

    for r in range(_K1_DEPTH):
        fire(r)

    def ring(r, _):
        pltpu.make_async_copy(ones_r, parts.at[idx2.at[r]], semh).wait()

        @pl.when(r + _K1_DEPTH < nrows)
        def _():
            fire(r + _K1_DEPTH)

        return 0

    lax.fori_loop(0, nrows, ring, 0)

    plsc.subcore_barrier()

    for t in range(NS):
        pltpu.async_copy(
            parts.at[pl.ds(t * _K1_PRIV + s * _K1_CMB, _K1_CMB)],
            cbuf.at[pl.ds(t * _K1_CMB, _K1_CMB)], semh)
    for t in range(NS):
        pltpu.make_async_copy(
            parts.at[pl.ds(t * _K1_PRIV + s * _K1_CMB, _K1_CMB)],
            cbuf.at[pl.ds(t * _K1_CMB, _K1_CMB)], semh).wait()

    def cmb(g, _):
        v = cbuf[pl.ds(g * LANES, LANES)]
        for t in range(1, NS):
            v = v + cbuf[pl.ds(t * _K1_CMB + g * LANES, LANES)]
        dbuf[pl.ds(g * LANES, LANES)] = v
        return 0

    lax.fori_loop(0, _K1_CMB // LANES, cmb, 0)

    @pl.when(s < NS - 1)
    def _():
        pltpu.sync_copy(dbuf, degs_hbm.at[pl.ds(c * NP + s * _K1_CMB,
                                                _K1_CMB)])

    @pl.when(s == NS - 1)
    def _():
        pltpu.sync_copy(
            dbuf.at[pl.ds(0, _K1_LAST)],
            degs_hbm.at[pl.ds(c * NP + (NS - 1) * _K1_CMB, _K1_LAST)])


def _degrees(edges2d):
    return pl.kernel(
        _deg_body,
        out_type=jax.ShapeDtypeStruct((2 * NP,), jnp.float32),
        mesh=_mesh(),
        scratch_types=[
            pltpu.VMEM((_K1_RT + 1, _K1_CHUNK), jnp.int32),
            pltpu.VMEM((_K1_CHUNK,), jnp.float32),
            pltpu.VMEM((_K1_CMB,), jnp.float32),
            pltpu.VMEM((_K1_PRIV,), jnp.float32),
            pltpu.SemaphoreType.DMA,
            pltpu.VMEM_SHARED((NS * _K1_PRIV,), jnp.float32),
        ],
        compiler_params=_SC_PARAMS,
    )(edges2d)


_AGG_B = 128
_EROWS = E // _AGG_B
_AGG_RT = _EROWS // (NC * NS)
_AGG_XTRA = _EROWS - _AGG_RT * NC * NS
_AGG_RPT = NP // NS
_AGG_OCH = 64


def _agg_body(table_hbm, src_hbm, dst_hbm, part_hbm, sidx_a, sidx_b,
              didx_a, didx_b, rows_a, rows_b, sem_a, sem_b, ssem_a, ssem_b,
              acc):
    c = lax.axis_index("c")
    s = lax.axis_index("s")
    wid = s * NC + c

    zv = jnp.zeros((LANES,), jnp.float32)

    def zr(r, _):
        for k in range(H // LANES):
            rows_a[r, pl.ds(k * LANES, LANES)] = zv
        return 0

    lax.fori_loop(0, _AGG_OCH, zr, 0)
    for k in range(_AGG_RPT // _AGG_OCH):
        pltpu.async_copy(rows_a.at[pl.ds(0, _AGG_OCH)],
                         acc.at[pl.ds(s * _AGG_RPT + k * _AGG_OCH, _AGG_OCH)],
                         sem_a)
    for k in range(_AGG_RPT // _AGG_OCH):
        pltpu.make_async_copy(
            rows_a.at[pl.ds(0, _AGG_OCH)],
            acc.at[pl.ds(s * _AGG_RPT + k * _AGG_OCH, _AGG_OCH)],
            sem_a).wait()
    plsc.subcore_barrier()

    e0 = wid * _AGG_RT * _AGG_B

    def eoff(j):
        return e0 + j * _AGG_B

    pltpu.sync_copy(src_hbm.at[pl.ds(eoff(0), _AGG_B)], sidx_a)
    pltpu.sync_copy(dst_hbm.at[pl.ds(eoff(0), _AGG_B)], didx_a.at[0])
    pltpu.async_copy(table_hbm.at[sidx_a], rows_a, sem_a)

    def pair(k, _):
        j0 = 2 * k
        j1 = j0 + 1
        pltpu.sync_copy(src_hbm.at[pl.ds(eoff(j1), _AGG_B)], sidx_b)
        pltpu.make_async_copy(table_hbm.at[sidx_a], rows_a, sem_a).wait()

        @pl.when(k > 0)
        def _():
            pltpu.make_async_copy(rows_b, acc.at[didx_b.at[0]], ssem_b).wait()

        pltpu.async_copy(table_hbm.at[sidx_b], rows_b, sem_b)
        pltpu.sync_copy(dst_hbm.at[pl.ds(eoff(j1), _AGG_B)], didx_b.at[0])
        pltpu.async_copy(rows_a, acc.at[didx_a.at[0]], ssem_a, add=True)

        @pl.when(j0 + 2 < _AGG_RT)
        def _():
            pltpu.sync_copy(src_hbm.at[pl.ds(eoff(j0 + 2), _AGG_B)], sidx_a)

        pltpu.make_async_copy(table_hbm.at[sidx_b], rows_b, sem_b).wait()
        pltpu.make_async_copy(rows_a, acc.at[didx_a.at[0]], ssem_a).wait()

        @pl.when(j0 + 2 < _AGG_RT)
        def _():
            pltpu.async_copy(table_hbm.at[sidx_a], rows_a, sem_a)
            pltpu.sync_copy(dst_hbm.at[pl.ds(eoff(j0 + 2), _AGG_B)],
                            didx_a.at[0])

        pltpu.async_copy(rows_b, acc.at[didx_b.at[0]], ssem_b, add=True)
        return 0

    lax.fori_loop(0, _AGG_RT // 2, pair, 0)
    pltpu.make_async_copy(rows_b, acc.at[didx_b.at[0]], ssem_b).wait()

    @pl.when(wid < _AGG_XTRA)
    def _():
        xe = (_EROWS - _AGG_XTRA + wid) * _AGG_B
        pltpu.sync_copy(src_hbm.at[pl.ds(xe, _AGG_B)], sidx_a)
        pltpu.sync_copy(dst_hbm.at[pl.ds(xe, _AGG_B)], didx_a.at[0])
        pltpu.sync_copy(table_hbm.at[sidx_a], rows_a)
        pltpu.sync_copy(rows_a, acc.at[didx_a.at[0]], add=True)

    plsc.subcore_barrier()

    nch = _AGG_RPT // _AGG_OCH

    def obuf(k):
        b = rows_a if k % 2 == 0 else rows_b
        return b.at[pl.ds(0, _AGG_OCH)]

    def isem(k):
        return sem_a if k % 2 == 0 else sem_b

    def osem(k):
        return ssem_a if k % 2 == 0 else ssem_b

    def rr(k):
        return s * _AGG_RPT + k * _AGG_OCH

    pltpu.async_copy(acc.at[pl.ds(rr(0), _AGG_OCH)], obuf(0), isem(0))
    pltpu.async_copy(acc.at[pl.ds(rr(1), _AGG_OCH)], obuf(1), isem(1))
    for k in range(nch):
        pltpu.make_async_copy(acc.at[pl.ds(rr(k), _AGG_OCH)], obuf(k),
                              isem(k)).wait()
        pltpu.async_copy(obuf(k), part_hbm.at[c, pl.ds(rr(k), _AGG_OCH)],
                         osem(k))
        if k + 2 < nch:
            pltpu.make_async_copy(obuf(k),
                                  part_hbm.at[c, pl.ds(rr(k), _AGG_OCH)],
                                  osem(k)).wait()
            pltpu.async_copy(acc.at[pl.ds(rr(k + 2), _AGG_OCH)], obuf(k + 2),
                             isem(k + 2))
    for k in (nch - 2, nch - 1):
        pltpu.make_async_copy(obuf(k), part_hbm.at[c, pl.ds(rr(k), _AGG_OCH)],
                              osem(k)).wait()


def _agg(table, src, dst):
    return pl.kernel(
        _agg_body,
        out_type=jax.ShapeDtypeStruct((NC, NP, H), jnp.float32),
        mesh=_mesh(),
        scratch_types=[
            pltpu.VMEM((_AGG_B,), jnp.int32),
            pltpu.VMEM((_AGG_B,), jnp.int32),
            pltpu.VMEM((1, _AGG_B), jnp.int32),
            pltpu.VMEM((1, _AGG_B), jnp.int32),
            pltpu.VMEM((_AGG_B, H), jnp.float32),
            pltpu.VMEM((_AGG_B, H), jnp.float32),
            pltpu.SemaphoreType.DMA,
            pltpu.SemaphoreType.DMA,
            pltpu.SemaphoreType.DMA,
            pltpu.SemaphoreType.DMA,
            pltpu.VMEM_SHARED((NP, H), jnp.float32),
        ],
        compiler_params=_SC_PARAMS,
    )(table, src, dst)


_TB = 1024


def _norm_from_deg(d):
    return jnp.where(d > 0, lax.rsqrt(jnp.maximum(d, 1e-12)), 0.0)


def _mm1_body(x_ref, w_ref, dego_ref, degi_ref, o_ref, no_ref, ni_ref):
    no = _norm_from_deg(dego_ref[...])
    ni = _norm_from_deg(degi_ref[...])
    no_ref[...] = no
    ni_ref[...] = ni
    o_ref[...] = (
        jnp.dot(x_ref[...], w_ref[...], preferred_element_type=jnp.float32)
        * no
    )


def _mm1(xp, W1, dego2d, degi2d):
    return pl.pallas_call(
        _mm1_body,
        grid=(NP // _TB,),
        in_specs=[
            pl.BlockSpec((_TB, D), lambda i: (i, 0)),
            pl.BlockSpec((D, H), lambda i: (0, 0)),
            pl.BlockSpec((_TB, 1), lambda i: (i, 0)),
            pl.BlockSpec((_TB, 1), lambda i: (i, 0)),
        ],
        out_specs=[
            pl.BlockSpec((_TB, H), lambda i: (i, 0)),
            pl.BlockSpec((_TB, 1), lambda i: (i, 0)),
            pl.BlockSpec((_TB, 1), lambda i: (i, 0)),
        ],
        out_shape=[
            jax.ShapeDtypeStruct((NP, H), jnp.float32),
            jax.ShapeDtypeStruct((NP, 1), jnp.float32),
            jax.ShapeDtypeStruct((NP, 1), jnp.float32),
        ],
    )(xp, W1, dego2d, degi2d)


def _mid_body(p_ref, ni_ref, b1_ref, w2_ref, no_ref, o_ref):
    sacc = p_ref[0] + p_ref[1]
    h = jnp.maximum(sacc * ni_ref[...] + b1_ref[...], 0.0)
    o_ref[...] = (
        jnp.dot(h, w2_ref[...], preferred_element_type=jnp.float32)
        * no_ref[...]
    )


def _mid(p, ni2d, b1r, W2p, no2d):
    return pl.pallas_call(
        _mid_body,
        grid=(NP // _TB,),
        in_specs=[
            pl.BlockSpec((NC, _TB, H), lambda i: (0, i, 0)),
            pl.BlockSpec((_TB, 1), lambda i: (i, 0)),
            pl.BlockSpec((1, H), lambda i: (0, 0)),
            pl.BlockSpec((H, H), lambda i: (0, 0)),
            pl.BlockSpec((_TB, 1), lambda i: (i, 0)),
        ],
        out_specs=pl.BlockSpec((_TB, H), lambda i: (i, 0)),
        out_shape=jax.ShapeDtypeStruct((NP, H), jnp.float32),
    )(p, ni2d, b1r, W2p, no2d)


def _fin_body(q_ref, ni_ref, b2_ref, o_ref):
    o_ref[...] = (q_ref[0] + q_ref[1]) * ni_ref[...] + b2_ref[...]


def _fin(q, ni2d, b2r):
    return pl.pallas_call(
        _fin_body,
        grid=(NP // _TB,),
        in_specs=[
            pl.BlockSpec((NC, _TB, H), lambda i: (0, i, 0)),
            pl.BlockSpec((_TB, 1), lambda i: (i, 0)),
            pl.BlockSpec((1, H), lambda i: (0, 0)),
        ],
        out_specs=pl.BlockSpec((_TB, H), lambda i: (i, 0)),
        out_shape=jax.ShapeDtypeStruct((NP, H), jnp.float32),
    )(q, ni2d, b2r)


def kernel(in_feat, edge_index, W1, b1, W2, b2):
    src = edge_index[0]
    dst = edge_index[1]
    xp = jnp.zeros((NP, D), in_feat.dtype).at[:N].set(in_feat)
    W2p = jnp.zeros((H, H), W2.dtype).at[:, :C].set(W2)
    b2p = jnp.zeros((1, H), b2.dtype).at[0, :C].set(b2)

    degs = _degrees(edge_index.reshape(2 * E // 128, 128))
    xw, no2d, ni2d = _mm1(xp, W1, degs[:NP].reshape(NP, 1),
                          degs[NP:].reshape(NP, 1))
    p = _agg(xw, src, dst)
    t2 = _mid(p, ni2d, b1.reshape(1, H), W2p, no2d)
    q = _agg(t2, src, dst)
    out = _fin(q, ni2d, b2p)
    return out[:N, :C]


# speedup vs baseline: 11.9916x; 1.0141x over previous
import functools

import jax
import jax.numpy as jnp
from jax import lax
from jax.experimental import pallas as pl
from jax.experimental.pallas import tpu as pltpu
from jax.experimental.pallas import tpu_sc as plsc

N = 10000
NP = 10240
E = 320000
D = 128
H = 128
C = 16

NC = 2
NS = 16
LANES = 16

_SC_PARAMS = pltpu.CompilerParams(use_tc_tiling_on_sc=False)


def _mesh():
    return plsc.VectorSubcoreMesh(core_axis_name="c", subcore_axis_name="s")


_K1_CHUNK = 128
_K1_ROWS = E // _K1_CHUNK
_K1_RT = _K1_ROWS // NS
_K1_XTRA = _K1_ROWS - _K1_RT * NS
_K1_CMB = 656
_K1_PRIV = _K1_CMB * NS
_K1_LAST = NP - (NS - 1) * _K1_CMB
_K1_DEPTH = 8


def _deg_body(edges_hbm, degs_hbm, idx2, ones_r, dbuf, cbuf, semh, parts):
    c = lax.axis_index("c")
    s = lax.axis_index("s")

    zv = jnp.zeros((LANES,), jnp.float32)
    ov = jnp.ones((LANES,), jnp.float32)

    def fo(r, _):
        ones_r[pl.ds(r * LANES, LANES)] = ov
        return 0

    lax.fori_loop(0, _K1_CHUNK // LANES, fo, 0)

    def fz(r, _):
        dbuf[pl.ds(r * LANES, LANES)] = zv
        return 0

    lax.fori_loop(0, _K1_CMB // LANES, fz, 0)

    for t in range(NS):
        pltpu.async_copy(
            dbuf, parts.at[pl.ds(s * _K1_PRIV + t * _K1_CMB, _K1_CMB)], semh)
    for t in range(NS):
        pltpu.make_async_copy(
            dbuf, parts.at[pl.ds(s * _K1_PRIV + t * _K1_CMB, _K1_CMB)],
            semh).wait()

    r0 = c * _K1_ROWS + s * _K1_RT
    pltpu.sync_copy(edges_hbm.at[pl.ds(r0, _K1_RT)], idx2.at[pl.ds(0, _K1_RT)])

    @pl.when(s < _K1_XTRA)
    def _():
        xr = c * _K1_ROWS + _K1_ROWS - _K1_XTRA + s
        pltpu.sync_copy(edges_hbm.at[xr], idx2.at[_K1_RT])

    shift = s * _K1_PRIV

    def shift_row(r, _):
        for g in range(_K1_CHUNK // LANES):
            idx2[r, pl.ds(g * LANES, LANES)] = (
                idx2[r, pl.ds(g * LANES, LANES)] + shift)
        return 0

    lax.fori_loop(0, _K1_RT + 1, shift_row, 0)

    nrows = jnp.where(s < _K1_XTRA, _K1_RT + 1, _K1_RT)

    def fire(r):
        pltpu.async_copy(ones_r, parts.at[idx2.at[r]], semh, add=True)

    for r in range(_K1_DEPTH):
        fire(r)

    def ring(r, _):
        pltpu.make_async_copy(ones_r, parts.at[idx2.at[r]], semh).wait()

        @pl.when(r + _K1_DEPTH < nrows)
        def _():
            fire(r + _K1_DEPTH)

        return 0

    lax.fori_loop(0, nrows, ring, 0)

    plsc.subcore_barrier()

    for t in range(NS):
        pltpu.async_copy(
            parts.at[pl.ds(t * _K1_PRIV + s * _K1_CMB, _K1_CMB)],
            cbuf.at[pl.ds(t * _K1_CMB, _K1_CMB)], semh)
    for t in range(NS):
        pltpu.make_async_copy(
            parts.at[pl.ds(t * _K1_PRIV + s * _K1_CMB, _K1_CMB)],
            cbuf.at[pl.ds(t * _K1_CMB, _K1_CMB)], semh).wait()

    def cmb(g, _):
        v = cbuf[pl.ds(g * LANES, LANES)]
        for t in range(1, NS):
            v = v + cbuf[pl.ds(t * _K1_CMB + g * LANES, LANES)]
        dbuf[pl.ds(g * LANES, LANES)] = v
        return 0

    lax.fori_loop(0, _K1_CMB // LANES, cmb, 0)

    @pl.when(s < NS - 1)
    def _():
        pltpu.sync_copy(dbuf, degs_hbm.at[pl.ds(c * NP + s * _K1_CMB,
                                                _K1_CMB)])

    @pl.when(s == NS - 1)
    def _():
        pltpu.sync_copy(
            dbuf.at[pl.ds(0, _K1_LAST)],
            degs_hbm.at[pl.ds(c * NP + (NS - 1) * _K1_CMB, _K1_LAST)])


def _degrees(edges2d):
    return pl.kernel(
        _deg_body,
        out_type=jax.ShapeDtypeStruct((2 * NP,), jnp.float32),
        mesh=_mesh(),
        scratch_types=[
            pltpu.VMEM((_K1_RT + 1, _K1_CHUNK), jnp.int32),
            pltpu.VMEM((_K1_CHUNK,), jnp.float32),
            pltpu.VMEM((_K1_CMB,), jnp.float32),
            pltpu.VMEM((_K1_PRIV,), jnp.float32),
            pltpu.SemaphoreType.DMA,
            pltpu.VMEM_SHARED((NS * _K1_PRIV,), jnp.float32),
        ],
        compiler_params=_SC_PARAMS,
    )(edges2d)


_AGG_B = 128
_EROWS = E // _AGG_B
_AGG_RT = _EROWS // (NC * NS)
_AGG_XTRA = _EROWS - _AGG_RT * NC * NS
_AGG_RPT = NP // NS
_AGG_OCH = 64


def _agg_body(table_hbm, edges_hbm, part_hbm, sidx_a, sidx_b,
              didx_a, didx_b, rows_a, rows_b, sem_a, sem_b, ssem_a, ssem_b,
              acc):
    c = lax.axis_index("c")
    s = lax.axis_index("s")
    wid = s * NC + c

    zv = jnp.zeros((LANES,), jnp.float32)

    def zr(r, _):
        for k in range(H // LANES):
            rows_a[r, pl.ds(k * LANES, LANES)] = zv
        return 0

    lax.fori_loop(0, _AGG_OCH, zr, 0)
    for k in range(_AGG_RPT // _AGG_OCH):
        pltpu.async_copy(rows_a.at[pl.ds(0, _AGG_OCH)],
                         acc.at[pl.ds(s * _AGG_RPT + k * _AGG_OCH, _AGG_OCH)],
                         sem_a)
    for k in range(_AGG_RPT // _AGG_OCH):
        pltpu.make_async_copy(
            rows_a.at[pl.ds(0, _AGG_OCH)],
            acc.at[pl.ds(s * _AGG_RPT + k * _AGG_OCH, _AGG_OCH)],
            sem_a).wait()
    plsc.subcore_barrier()

    r0 = wid * _AGG_RT

    def srow(j):
        return r0 + j

    def drow(j):
        return _EROWS + r0 + j

    pltpu.sync_copy(edges_hbm.at[srow(0)], sidx_a)
    pltpu.sync_copy(edges_hbm.at[drow(0)], didx_a.at[0])
    pltpu.async_copy(table_hbm.at[sidx_a], rows_a, sem_a)

    def pair(k, _):
        j0 = 2 * k
        j1 = j0 + 1
        pltpu.sync_copy(edges_hbm.at[srow(j1)], sidx_b)
        pltpu.make_async_copy(table_hbm.at[sidx_a], rows_a, sem_a).wait()

        @pl.when(k > 0)
        def _():
            pltpu.make_async_copy(rows_b, acc.at[didx_b.at[0]], ssem_b).wait()

        pltpu.async_copy(table_hbm.at[sidx_b], rows_b, sem_b)
        pltpu.sync_copy(edges_hbm.at[drow(j1)], didx_b.at[0])
        pltpu.async_copy(rows_a, acc.at[didx_a.at[0]], ssem_a, add=True)

        @pl.when(j0 + 2 < _AGG_RT)
        def _():
            pltpu.sync_copy(edges_hbm.at[srow(j0 + 2)], sidx_a)

        pltpu.make_async_copy(table_hbm.at[sidx_b], rows_b, sem_b).wait()
        pltpu.make_async_copy(rows_a, acc.at[didx_a.at[0]], ssem_a).wait()

        @pl.when(j0 + 2 < _AGG_RT)
        def _():
            pltpu.async_copy(table_hbm.at[sidx_a], rows_a, sem_a)
            pltpu.sync_copy(edges_hbm.at[drow(j0 + 2)], didx_a.at[0])

        pltpu.async_copy(rows_b, acc.at[didx_b.at[0]], ssem_b, add=True)
        return 0

    lax.fori_loop(0, _AGG_RT // 2, pair, 0)
    pltpu.make_async_copy(rows_b, acc.at[didx_b.at[0]], ssem_b).wait()

    @pl.when(wid < _AGG_XTRA)
    def _():
        xr = _EROWS - _AGG_XTRA + wid
        pltpu.sync_copy(edges_hbm.at[xr], sidx_a)
        pltpu.sync_copy(edges_hbm.at[_EROWS + xr], didx_a.at[0])
        pltpu.sync_copy(table_hbm.at[sidx_a], rows_a)
        pltpu.sync_copy(rows_a, acc.at[didx_a.at[0]], add=True)

    plsc.subcore_barrier()

    nch = _AGG_RPT // _AGG_OCH

    def obuf(k):
        b = rows_a if k % 2 == 0 else rows_b
        return b.at[pl.ds(0, _AGG_OCH)]

    def isem(k):
        return sem_a if k % 2 == 0 else sem_b

    def osem(k):
        return ssem_a if k % 2 == 0 else ssem_b

    def rr(k):
        return s * _AGG_RPT + k * _AGG_OCH

    pltpu.async_copy(acc.at[pl.ds(rr(0), _AGG_OCH)], obuf(0), isem(0))
    pltpu.async_copy(acc.at[pl.ds(rr(1), _AGG_OCH)], obuf(1), isem(1))
    for k in range(nch):
        pltpu.make_async_copy(acc.at[pl.ds(rr(k), _AGG_OCH)], obuf(k),
                              isem(k)).wait()
        pltpu.async_copy(obuf(k), part_hbm.at[c, pl.ds(rr(k), _AGG_OCH)],
                         osem(k))
        if k + 2 < nch:
            pltpu.make_async_copy(obuf(k),
                                  part_hbm.at[c, pl.ds(rr(k), _AGG_OCH)],
                                  osem(k)).wait()
            pltpu.async_copy(acc.at[pl.ds(rr(k + 2), _AGG_OCH)], obuf(k + 2),
                             isem(k + 2))
    for k in (nch - 2, nch - 1):
        pltpu.make_async_copy(obuf(k), part_hbm.at[c, pl.ds(rr(k), _AGG_OCH)],
                              osem(k)).wait()


def _agg(table, edges2d):
    return pl.kernel(
        _agg_body,
        out_type=jax.ShapeDtypeStruct((NC, NP, H), jnp.float32),
        mesh=_mesh(),
        scratch_types=[
            pltpu.VMEM((_AGG_B,), jnp.int32),
            pltpu.VMEM((_AGG_B,), jnp.int32),
            pltpu.VMEM((1, _AGG_B), jnp.int32),
            pltpu.VMEM((1, _AGG_B), jnp.int32),
            pltpu.VMEM((_AGG_B, H), jnp.float32),
            pltpu.VMEM((_AGG_B, H), jnp.float32),
            pltpu.SemaphoreType.DMA,
            pltpu.SemaphoreType.DMA,
            pltpu.SemaphoreType.DMA,
            pltpu.SemaphoreType.DMA,
            pltpu.VMEM_SHARED((NP, H), jnp.float32),
        ],
        compiler_params=_SC_PARAMS,
    )(table, edges2d)


_TB = 1024


def _norm_from_deg(d):
    return jnp.where(d > 0, lax.rsqrt(jnp.maximum(d, 1e-12)), 0.0)


def _mm1_body(x_ref, w_ref, dego_ref, degi_ref, o_ref, no_ref, ni_ref):
    no = _norm_from_deg(dego_ref[...])
    ni = _norm_from_deg(degi_ref[...])
    no_ref[...] = no
    ni_ref[...] = ni
    o_ref[...] = (
        jnp.dot(x_ref[...], w_ref[...], preferred_element_type=jnp.float32)
        * no
    )


def _mm1(x, W1, dego2d, degi2d):
    return pl.pallas_call(
        _mm1_body,
        grid=(NP // _TB,),
        in_specs=[
            pl.BlockSpec((_TB, D), lambda i: (i, 0)),
            pl.BlockSpec((D, H), lambda i: (0, 0)),
            pl.BlockSpec((_TB, 1), lambda i: (i, 0)),
            pl.BlockSpec((_TB, 1), lambda i: (i, 0)),
        ],
        out_specs=[
            pl.BlockSpec((_TB, H), lambda i: (i, 0)),
            pl.BlockSpec((_TB, 1), lambda i: (i, 0)),
            pl.BlockSpec((_TB, 1), lambda i: (i, 0)),
        ],
        out_shape=[
            jax.ShapeDtypeStruct((NP, H), jnp.float32),
            jax.ShapeDtypeStruct((NP, 1), jnp.float32),
            jax.ShapeDtypeStruct((NP, 1), jnp.float32),
        ],
    )(x, W1, dego2d, degi2d)


def _mid_body(p_ref, ni_ref, b1_ref, w2_ref, no_ref, o_ref):
    sacc = p_ref[0] + p_ref[1]
    h = jnp.maximum(sacc * ni_ref[...] + b1_ref[...], 0.0)
    o_ref[...] = (
        jnp.dot(h, w2_ref[...], preferred_element_type=jnp.float32)
        * no_ref[...]
    )


def _mid(p, ni2d, b1r, W2p, no2d):
    return pl.pallas_call(
        _mid_body,
        grid=(NP // _TB,),
        in_specs=[
            pl.BlockSpec((NC, _TB, H), lambda i: (0, i, 0)),
            pl.BlockSpec((_TB, 1), lambda i: (i, 0)),
            pl.BlockSpec((1, H), lambda i: (0, 0)),
            pl.BlockSpec((H, H), lambda i: (0, 0)),
            pl.BlockSpec((_TB, 1), lambda i: (i, 0)),
        ],
        out_specs=pl.BlockSpec((_TB, H), lambda i: (i, 0)),
        out_shape=jax.ShapeDtypeStruct((NP, H), jnp.float32),
    )(p, ni2d, b1r, W2p, no2d)


def _fin_body(q_ref, ni_ref, b2_ref, o_ref):
    o_ref[...] = (q_ref[0] + q_ref[1]) * ni_ref[...] + b2_ref[...]


def _fin(q, ni2d, b2r):
    return pl.pallas_call(
        _fin_body,
        grid=(NP // _TB,),
        in_specs=[
            pl.BlockSpec((NC, _TB, H), lambda i: (0, i, 0)),
            pl.BlockSpec((_TB, 1), lambda i: (i, 0)),
            pl.BlockSpec((1, H), lambda i: (0, 0)),
        ],
        out_specs=pl.BlockSpec((_TB, H), lambda i: (i, 0)),
        out_shape=jax.ShapeDtypeStruct((NP, H), jnp.float32),
    )(q, ni2d, b2r)


def kernel(in_feat, edge_index, W1, b1, W2, b2):
    edges2d = edge_index.reshape(2 * E // _AGG_B, _AGG_B)
    W2p = jnp.zeros((H, H), W2.dtype).at[:, :C].set(W2)
    b2p = jnp.zeros((1, H), b2.dtype).at[0, :C].set(b2)

    degs = _degrees(edges2d)
    xw, no2d, ni2d = _mm1(in_feat, W1, degs[:NP].reshape(NP, 1),
                          degs[NP:].reshape(NP, 1))
    p = _agg(xw, edges2d)
    t2 = _mid(p, ni2d, b1.reshape(1, H), W2p, no2d)
    q = _agg(t2, edges2d)
    out = _fin(q, ni2d, b2p)
    return out[:N, :C]


# speedup vs baseline: 12.0574x; 1.0055x over previous
import functools

import jax
import jax.numpy as jnp
from jax import lax
from jax.experimental import pallas as pl
from jax.experimental.pallas import tpu as pltpu
from jax.experimental.pallas import tpu_sc as plsc

N = 10000
NP = 10240
E = 320000
D = 128
H = 128
C = 16

NC = 2
NS = 16
LANES = 16

_SC_PARAMS = pltpu.CompilerParams(use_tc_tiling_on_sc=False)


def _mesh():
    return plsc.VectorSubcoreMesh(core_axis_name="c", subcore_axis_name="s")


_K1_CHUNK = 128
_K1_ROWS = E // _K1_CHUNK
_K1_RT = _K1_ROWS // NS
_K1_XTRA = _K1_ROWS - _K1_RT * NS
_K1_CMB = 656
_K1_PRIV = _K1_CMB * NS
_K1_LAST = NP - (NS - 1) * _K1_CMB
_K1_DEPTH = 8


def _deg_body(edges_hbm, degs_hbm, idx2, ones_r, dbuf, cbuf, semh, parts):
    c = lax.axis_index("c")
    s = lax.axis_index("s")

    zv = jnp.zeros((LANES,), jnp.float32)
    ov = jnp.ones((LANES,), jnp.float32)

    def fo(r, _):
        ones_r[pl.ds(r * LANES, LANES)] = ov
        return 0

    lax.fori_loop(0, _K1_CHUNK // LANES, fo, 0)

    def fz(r, _):
        dbuf[pl.ds(r * LANES, LANES)] = zv
        return 0

    lax.fori_loop(0, _K1_CMB // LANES, fz, 0)

    for t in range(NS):
        pltpu.async_copy(
            dbuf, parts.at[pl.ds(s * _K1_PRIV + t * _K1_CMB, _K1_CMB)], semh)
    for t in range(NS):
        pltpu.make_async_copy(
            dbuf, parts.at[pl.ds(s * _K1_PRIV + t * _K1_CMB, _K1_CMB)],
            semh).wait()

    r0 = c * _K1_ROWS + s * _K1_RT
    pltpu.sync_copy(edges_hbm.at[pl.ds(r0, _K1_RT)], idx2.at[pl.ds(0, _K1_RT)])

    @pl.when(s < _K1_XTRA)
    def _():
        xr = c * _K1_ROWS + _K1_ROWS - _K1_XTRA + s
        pltpu.sync_copy(edges_hbm.at[xr], idx2.at[_K1_RT])

    shift = s * _K1_PRIV

    def shift_row(r, _):
        for g in range(_K1_CHUNK // LANES):
            idx2[r, pl.ds(g * LANES, LANES)] = (
                idx2[r, pl.ds(g * LANES, LANES)] + shift)
        return 0

    lax.fori_loop(0, _K1_RT + 1, shift_row, 0)

    nrows = jnp.where(s < _K1_XTRA, _K1_RT + 1, _K1_RT)

    def fire(r):
        pltpu.async_copy(ones_r, parts.at[idx2.at[r]], semh, add=True)

    for r in range(_K1_DEPTH):
        fire(r)

    def ring(r, _):
        pltpu.make_async_copy(ones_r, parts.at[idx2.at[r]], semh).wait()

        @pl.when(r + _K1_DEPTH < nrows)
        def _():
            fire(r + _K1_DEPTH)

        return 0

    lax.fori_loop(0, nrows, ring, 0)

    plsc.subcore_barrier()

    for t in range(NS):
        pltpu.async_copy(
            parts.at[pl.ds(t * _K1_PRIV + s * _K1_CMB, _K1_CMB)],
            cbuf.at[pl.ds(t * _K1_CMB, _K1_CMB)], semh)
    for t in range(NS):
        pltpu.make_async_copy(
            parts.at[pl.ds(t * _K1_PRIV + s * _K1_CMB, _K1_CMB)],
            cbuf.at[pl.ds(t * _K1_CMB, _K1_CMB)], semh).wait()

    def cmb(g, _):
        v = cbuf[pl.ds(g * LANES, LANES)]
        for t in range(1, NS):
            v = v + cbuf[pl.ds(t * _K1_CMB + g * LANES, LANES)]
        dbuf[pl.ds(g * LANES, LANES)] = v
        return 0

    lax.fori_loop(0, _K1_CMB // LANES, cmb, 0)

    @pl.when(s < NS - 1)
    def _():
        pltpu.sync_copy(dbuf, degs_hbm.at[pl.ds(c * NP + s * _K1_CMB,
                                                _K1_CMB)])

    @pl.when(s == NS - 1)
    def _():
        pltpu.sync_copy(
            dbuf.at[pl.ds(0, _K1_LAST)],
            degs_hbm.at[pl.ds(c * NP + (NS - 1) * _K1_CMB, _K1_LAST)])


def _degrees(edges2d):
    return pl.kernel(
        _deg_body,
        out_type=jax.ShapeDtypeStruct((2 * NP,), jnp.float32),
        mesh=_mesh(),
        scratch_types=[
            pltpu.VMEM((_K1_RT + 1, _K1_CHUNK), jnp.int32),
            pltpu.VMEM((_K1_CHUNK,), jnp.float32),
            pltpu.VMEM((_K1_CMB,), jnp.float32),
            pltpu.VMEM((_K1_PRIV,), jnp.float32),
            pltpu.SemaphoreType.DMA,
            pltpu.VMEM_SHARED((NS * _K1_PRIV,), jnp.float32),
        ],
        compiler_params=_SC_PARAMS,
    )(edges2d)


_AGG_B = 128
_EROWS = E // _AGG_B
_AGG_RT = _EROWS // (NC * NS)
_AGG_XTRA = _EROWS - _AGG_RT * NC * NS
_AGG_RPT = NP // NS
_AGG_OCH = 64


def _agg_body(table_hbm, edges_hbm, part_hbm, sidx_a, sidx_b,
              didx_a, didx_b, rows_a, rows_b, sem_a, sem_b, ssem_a, ssem_b,
              acc):
    c = lax.axis_index("c")
    s = lax.axis_index("s")
    wid = s * NC + c

    zv = jnp.zeros((LANES,), jnp.float32)

    def zr(r, _):
        for k in range(H // LANES):
            rows_a[r, pl.ds(k * LANES, LANES)] = zv
        return 0

    lax.fori_loop(0, _AGG_OCH, zr, 0)
    for k in range(_AGG_RPT // _AGG_OCH):
        pltpu.async_copy(rows_a.at[pl.ds(0, _AGG_OCH)],
                         acc.at[pl.ds(s * _AGG_RPT + k * _AGG_OCH, _AGG_OCH)],
                         sem_a)
    for k in range(_AGG_RPT // _AGG_OCH):
        pltpu.make_async_copy(
            rows_a.at[pl.ds(0, _AGG_OCH)],
            acc.at[pl.ds(s * _AGG_RPT + k * _AGG_OCH, _AGG_OCH)],
            sem_a).wait()
    plsc.subcore_barrier()

    r0 = wid * _AGG_RT

    def srow(j):
        return r0 + j

    def drow(j):
        return _EROWS + r0 + j

    pltpu.sync_copy(edges_hbm.at[srow(0)], sidx_a)
    pltpu.sync_copy(edges_hbm.at[drow(0)], didx_a.at[0])
    pltpu.async_copy(table_hbm.at[sidx_a], rows_a, sem_a)

    def pair(k, _):
        j0 = 2 * k
        j1 = j0 + 1
        pltpu.sync_copy(edges_hbm.at[srow(j1)], sidx_b)
        pltpu.make_async_copy(table_hbm.at[sidx_a], rows_a, sem_a).wait()

        @pl.when(k > 0)
        def _():
            pltpu.make_async_copy(rows_b, acc.at[didx_b.at[0]], ssem_b).wait()

        pltpu.async_copy(table_hbm.at[sidx_b], rows_b, sem_b)
        pltpu.sync_copy(edges_hbm.at[drow(j1)], didx_b.at[0])
        pltpu.async_copy(rows_a, acc.at[didx_a.at[0]], ssem_a, add=True)

        @pl.when(j0 + 2 < _AGG_RT)
        def _():
            pltpu.sync_copy(edges_hbm.at[srow(j0 + 2)], sidx_a)

        pltpu.make_async_copy(table_hbm.at[sidx_b], rows_b, sem_b).wait()
        pltpu.make_async_copy(rows_a, acc.at[didx_a.at[0]], ssem_a).wait()

        @pl.when(j0 + 2 < _AGG_RT)
        def _():
            pltpu.async_copy(table_hbm.at[sidx_a], rows_a, sem_a)
            pltpu.sync_copy(edges_hbm.at[drow(j0 + 2)], didx_a.at[0])

        pltpu.async_copy(rows_b, acc.at[didx_b.at[0]], ssem_b, add=True)
        return 0

    lax.fori_loop(0, _AGG_RT // 2, pair, 0)
    pltpu.make_async_copy(rows_b, acc.at[didx_b.at[0]], ssem_b).wait()

    @pl.when(wid < _AGG_XTRA)
    def _():
        xr = _EROWS - _AGG_XTRA + wid
        pltpu.sync_copy(edges_hbm.at[xr], sidx_a)
        pltpu.sync_copy(edges_hbm.at[_EROWS + xr], didx_a.at[0])
        pltpu.sync_copy(table_hbm.at[sidx_a], rows_a)
        pltpu.sync_copy(rows_a, acc.at[didx_a.at[0]], add=True)

    plsc.subcore_barrier()

    nch = _AGG_RPT // _AGG_OCH

    def obuf(k):
        b = rows_a if k % 2 == 0 else rows_b
        return b.at[pl.ds(0, _AGG_OCH)]

    def isem(k):
        return sem_a if k % 2 == 0 else sem_b

    def osem(k):
        return ssem_a if k % 2 == 0 else ssem_b

    def rr(k):
        return s * _AGG_RPT + k * _AGG_OCH

    pltpu.async_copy(acc.at[pl.ds(rr(0), _AGG_OCH)], obuf(0), isem(0))
    pltpu.async_copy(acc.at[pl.ds(rr(1), _AGG_OCH)], obuf(1), isem(1))
    for k in range(nch):
        pltpu.make_async_copy(acc.at[pl.ds(rr(k), _AGG_OCH)], obuf(k),
                              isem(k)).wait()
        pltpu.async_copy(obuf(k), part_hbm.at[c, pl.ds(rr(k), _AGG_OCH)],
                         osem(k))
        if k + 2 < nch:
            pltpu.make_async_copy(obuf(k),
                                  part_hbm.at[c, pl.ds(rr(k), _AGG_OCH)],
                                  osem(k)).wait()
            pltpu.async_copy(acc.at[pl.ds(rr(k + 2), _AGG_OCH)], obuf(k + 2),
                             isem(k + 2))
    for k in (nch - 2, nch - 1):
        pltpu.make_async_copy(obuf(k), part_hbm.at[c, pl.ds(rr(k), _AGG_OCH)],
                              osem(k)).wait()


def _agg(table, edges2d):
    return pl.kernel(
        _agg_body,
        out_type=jax.ShapeDtypeStruct((NC, NP, H), jnp.float32),
        mesh=_mesh(),
        scratch_types=[
            pltpu.VMEM((_AGG_B,), jnp.int32),
            pltpu.VMEM((_AGG_B,), jnp.int32),
            pltpu.VMEM((1, _AGG_B), jnp.int32),
            pltpu.VMEM((1, _AGG_B), jnp.int32),
            pltpu.VMEM((_AGG_B, H), jnp.float32),
            pltpu.VMEM((_AGG_B, H), jnp.float32),
            pltpu.SemaphoreType.DMA,
            pltpu.SemaphoreType.DMA,
            pltpu.SemaphoreType.DMA,
            pltpu.SemaphoreType.DMA,
            pltpu.VMEM_SHARED((NP, H), jnp.float32),
        ],
        compiler_params=_SC_PARAMS,
    )(table, edges2d)


_TB = 1024


def _norm_from_deg(d):
    return jnp.where(d > 0, lax.rsqrt(jnp.maximum(d, 1e-12)), 0.0)


def _mm1_body(x_ref, w_ref, dego_ref, degi_ref, o_ref, no_ref, ni_ref):
    no = _norm_from_deg(dego_ref[...])
    ni = _norm_from_deg(degi_ref[...])
    no_ref[...] = no
    ni_ref[...] = ni
    o_ref[...] = (
        jnp.dot(x_ref[...], w_ref[...], preferred_element_type=jnp.float32)
        * no
    )


def _mm1(x, W1, dego2d, degi2d):
    return pl.pallas_call(
        _mm1_body,
        grid=(NP // _TB,),
        in_specs=[
            pl.BlockSpec((_TB, D), lambda i: (i, 0)),
            pl.BlockSpec((D, H), lambda i: (0, 0)),
            pl.BlockSpec((_TB, 1), lambda i: (i, 0)),
            pl.BlockSpec((_TB, 1), lambda i: (i, 0)),
        ],
        out_specs=[
            pl.BlockSpec((_TB, H), lambda i: (i, 0)),
            pl.BlockSpec((_TB, 1), lambda i: (i, 0)),
            pl.BlockSpec((_TB, 1), lambda i: (i, 0)),
        ],
        out_shape=[
            jax.ShapeDtypeStruct((NP, H), jnp.float32),
            jax.ShapeDtypeStruct((NP, 1), jnp.float32),
            jax.ShapeDtypeStruct((NP, 1), jnp.float32),
        ],
    )(x, W1, dego2d, degi2d)


def _mid_body(p_ref, ni_ref, b1_ref, w2_ref, no_ref, o_ref):
    sacc = p_ref[0] + p_ref[1]
    h = jnp.maximum(sacc * ni_ref[...] + b1_ref[...], 0.0)
    o_ref[...] = (
        jnp.dot(h, w2_ref[...], preferred_element_type=jnp.float32)
        * no_ref[...]
    )


def _mid(p, ni2d, b1r, W2p, no2d):
    return pl.pallas_call(
        _mid_body,
        grid=(NP // _TB,),
        in_specs=[
            pl.BlockSpec((NC, _TB, H), lambda i: (0, i, 0)),
            pl.BlockSpec((_TB, 1), lambda i: (i, 0)),
            pl.BlockSpec((1, H), lambda i: (0, 0)),
            pl.BlockSpec((H, H), lambda i: (0, 0)),
            pl.BlockSpec((_TB, 1), lambda i: (i, 0)),
        ],
        out_specs=pl.BlockSpec((_TB, H), lambda i: (i, 0)),
        out_shape=jax.ShapeDtypeStruct((NP, H), jnp.float32),
    )(p, ni2d, b1r, W2p, no2d)


def _fin_body(q_ref, ni_ref, b2_ref, o_ref):
    v = (q_ref[0] + q_ref[1]) * ni_ref[...] + b2_ref[...]
    o_ref[...] = v[:, :C]


def _fin(q, ni2d, b2r):
    nb = N // 10
    return pl.pallas_call(
        _fin_body,
        grid=(10,),
        in_specs=[
            pl.BlockSpec((NC, nb, H), lambda i: (0, i, 0)),
            pl.BlockSpec((nb, 1), lambda i: (i, 0)),
            pl.BlockSpec((1, H), lambda i: (0, 0)),
        ],
        out_specs=pl.BlockSpec((nb, C), lambda i: (i, 0)),
        out_shape=jax.ShapeDtypeStruct((N, C), jnp.float32),
    )(q, ni2d, b2r)


def kernel(in_feat, edge_index, W1, b1, W2, b2):
    edges2d = edge_index.reshape(2 * E // _AGG_B, _AGG_B)
    W2p = jnp.zeros((H, H), W2.dtype).at[:, :C].set(W2)
    b2p = jnp.zeros((1, H), b2.dtype).at[0, :C].set(b2)

    degs = _degrees(edges2d)
    xw, no2d, ni2d = _mm1(in_feat, W1, degs[:NP].reshape(NP, 1),
                          degs[NP:].reshape(NP, 1))
    p = _agg(xw, edges2d)
    t2 = _mid(p, ni2d, b1.reshape(1, H), W2p, no2d)
    q = _agg(t2, edges2d)
    return _fin(q, ni2d, b2p)


# speedup vs baseline: 12.0687x; 1.0009x over previous
import jax
import jax.numpy as jnp
from jax import lax
from jax.experimental import pallas as pl
from jax.experimental.pallas import tpu as pltpu
from jax.experimental.pallas import tpu_sc as plsc

N = 10000
NP = 10240
E = 320000
D = 128
H = 128
C = 16

NC = 2
NS = 16
LANES = 16

_SC_PARAMS = pltpu.CompilerParams(use_tc_tiling_on_sc=False)


def _mesh():
    return plsc.VectorSubcoreMesh(core_axis_name="c", subcore_axis_name="s")


_K1_CHUNK = 128
_K1_ROWS = E // _K1_CHUNK
_K1_RT = _K1_ROWS // NS
_K1_XTRA = _K1_ROWS - _K1_RT * NS
_K1_CMB = 656
_K1_PRIV = _K1_CMB * NS
_K1_LAST = NP - (NS - 1) * _K1_CMB
_K1_DEPTH = 8


def _deg_body(edges_hbm, degs_hbm, idx2, ones_r, dbuf, cbuf, semh, parts):
    c = lax.axis_index("c")
    s = lax.axis_index("s")

    zv = jnp.zeros((LANES,), jnp.float32)
    ov = jnp.ones((LANES,), jnp.float32)

    def fo(r, _):
        ones_r[pl.ds(r * LANES, LANES)] = ov
        return 0

    lax.fori_loop(0, _K1_CHUNK // LANES, fo, 0)

    def fz(r, _):
        dbuf[pl.ds(r * LANES, LANES)] = zv
        return 0

    lax.fori_loop(0, _K1_CMB // LANES, fz, 0)

    for t in range(NS):
        pltpu.async_copy(
            dbuf, parts.at[pl.ds(s * _K1_PRIV + t * _K1_CMB, _K1_CMB)], semh)
    for t in range(NS):
        pltpu.make_async_copy(
            dbuf, parts.at[pl.ds(s * _K1_PRIV + t * _K1_CMB, _K1_CMB)],
            semh).wait()

    r0 = c * _K1_ROWS + s * _K1_RT
    pltpu.sync_copy(edges_hbm.at[pl.ds(r0, _K1_RT)], idx2.at[pl.ds(0, _K1_RT)])

    @pl.when(s < _K1_XTRA)
    def _():
        xr = c * _K1_ROWS + _K1_ROWS - _K1_XTRA + s
        pltpu.sync_copy(edges_hbm.at[xr], idx2.at[_K1_RT])

    shift = s * _K1_PRIV

    def shift_row(r, _):
        for g in range(_K1_CHUNK // LANES):
            idx2[r, pl.ds(g * LANES, LANES)] = (
                idx2[r, pl.ds(g * LANES, LANES)] + shift)
        return 0

    lax.fori_loop(0, _K1_RT + 1, shift_row, 0)

    nrows = jnp.where(s < _K1_XTRA, _K1_RT + 1, _K1_RT)

    def fire(r):
        pltpu.async_copy(ones_r, parts.at[idx2.at[r]], semh, add=True)

    for r in range(_K1_DEPTH):
        fire(r)

    def ring(r, _):
        pltpu.make_async_copy(ones_r, parts.at[idx2.at[r]], semh).wait()

        @pl.when(r + _K1_DEPTH < nrows)
        def _():
            fire(r + _K1_DEPTH)

        return 0

    lax.fori_loop(0, nrows, ring, 0)

    plsc.subcore_barrier()

    for t in range(NS):
        pltpu.async_copy(
            parts.at[pl.ds(t * _K1_PRIV + s * _K1_CMB, _K1_CMB)],
            cbuf.at[pl.ds(t * _K1_CMB, _K1_CMB)], semh)
    for t in range(NS):
        pltpu.make_async_copy(
            parts.at[pl.ds(t * _K1_PRIV + s * _K1_CMB, _K1_CMB)],
            cbuf.at[pl.ds(t * _K1_CMB, _K1_CMB)], semh).wait()

    def cmb(g, _):
        v = cbuf[pl.ds(g * LANES, LANES)]
        for t in range(1, NS):
            v = v + cbuf[pl.ds(t * _K1_CMB + g * LANES, LANES)]
        dbuf[pl.ds(g * LANES, LANES)] = v
        return 0

    lax.fori_loop(0, _K1_CMB // LANES, cmb, 0)

    @pl.when(s < NS - 1)
    def _():
        pltpu.sync_copy(dbuf, degs_hbm.at[pl.ds(c * NP + s * _K1_CMB,
                                                _K1_CMB)])

    @pl.when(s == NS - 1)
    def _():
        pltpu.sync_copy(
            dbuf.at[pl.ds(0, _K1_LAST)],
            degs_hbm.at[pl.ds(c * NP + (NS - 1) * _K1_CMB, _K1_LAST)])


def _degrees(edges2d):
    return pl.kernel(
        _deg_body,
        out_type=jax.ShapeDtypeStruct((2 * NP,), jnp.float32),
        mesh=_mesh(),
        scratch_types=[
            pltpu.VMEM((_K1_RT + 1, _K1_CHUNK), jnp.int32),
            pltpu.VMEM((_K1_CHUNK,), jnp.float32),
            pltpu.VMEM((_K1_CMB,), jnp.float32),
            pltpu.VMEM((_K1_PRIV,), jnp.float32),
            pltpu.SemaphoreType.DMA,
            pltpu.VMEM_SHARED((NS * _K1_PRIV,), jnp.float32),
        ],
        compiler_params=_SC_PARAMS,
    )(edges2d)


_AGG_B = 128
_EROWS = E // _AGG_B
_AGG_RT = _EROWS // (NC * NS)
_AGG_XTRA = _EROWS - _AGG_RT * NC * NS
_AGG_RPT = NP // NS
_AGG_OCH = 64


def _agg_body(table_hbm, edges_hbm, part_hbm, sidx_a, sidx_b,
              didx_a, didx_b, rows_a, rows_b, sem_a, sem_b, ssem_a, ssem_b,
              acc):
    c = lax.axis_index("c")
    s = lax.axis_index("s")
    wid = s * NC + c

    zv = jnp.zeros((LANES,), jnp.float32)

    def zr(r, _):
        for k in range(H // LANES):
            rows_a[r, pl.ds(k * LANES, LANES)] = zv
        return 0

    lax.fori_loop(0, _AGG_OCH, zr, 0)
    for k in range(_AGG_RPT // _AGG_OCH):
        pltpu.async_copy(rows_a.at[pl.ds(0, _AGG_OCH)],
                         acc.at[pl.ds(s * _AGG_RPT + k * _AGG_OCH, _AGG_OCH)],
                         sem_a)
    for k in range(_AGG_RPT // _AGG_OCH):
        pltpu.make_async_copy(
            rows_a.at[pl.ds(0, _AGG_OCH)],
            acc.at[pl.ds(s * _AGG_RPT + k * _AGG_OCH, _AGG_OCH)],
            sem_a).wait()
    plsc.subcore_barrier()

    r0 = wid * _AGG_RT

    def srow(j):
        return r0 + j

    def drow(j):
        return _EROWS + r0 + j

    pltpu.sync_copy(edges_hbm.at[srow(0)], sidx_a)
    pltpu.sync_copy(edges_hbm.at[drow(0)], didx_a.at[0])
    pltpu.async_copy(table_hbm.at[sidx_a], rows_a, sem_a)

    def pair(k, _):
        j0 = 2 * k
        j1 = j0 + 1
        pltpu.sync_copy(edges_hbm.at[srow(j1)], sidx_b)
        pltpu.make_async_copy(table_hbm.at[sidx_a], rows_a, sem_a).wait()

        @pl.when(k > 0)
        def _():
            pltpu.make_async_copy(rows_b, acc.at[didx_b.at[0]], ssem_b).wait()

        pltpu.async_copy(table_hbm.at[sidx_b], rows_b, sem_b)
        pltpu.sync_copy(edges_hbm.at[drow(j1)], didx_b.at[0])
        pltpu.async_copy(rows_a, acc.at[didx_a.at[0]], ssem_a, add=True)

        @pl.when(j0 + 2 < _AGG_RT)
        def _():
            pltpu.sync_copy(edges_hbm.at[srow(j0 + 2)], sidx_a)

        pltpu.make_async_copy(table_hbm.at[sidx_b], rows_b, sem_b).wait()
        pltpu.make_async_copy(rows_a, acc.at[didx_a.at[0]], ssem_a).wait()

        @pl.when(j0 + 2 < _AGG_RT)
        def _():
            pltpu.async_copy(table_hbm.at[sidx_a], rows_a, sem_a)
            pltpu.sync_copy(edges_hbm.at[drow(j0 + 2)], didx_a.at[0])

        pltpu.async_copy(rows_b, acc.at[didx_b.at[0]], ssem_b, add=True)
        return 0

    lax.fori_loop(0, _AGG_RT // 2, pair, 0)
    pltpu.make_async_copy(rows_b, acc.at[didx_b.at[0]], ssem_b).wait()

    @pl.when(wid < _AGG_XTRA)
    def _():
        xr = _EROWS - _AGG_XTRA + wid
        pltpu.sync_copy(edges_hbm.at[xr], sidx_a)
        pltpu.sync_copy(edges_hbm.at[_EROWS + xr], didx_a.at[0])
        pltpu.sync_copy(table_hbm.at[sidx_a], rows_a)
        pltpu.sync_copy(rows_a, acc.at[didx_a.at[0]], add=True)

    plsc.subcore_barrier()

    nch = _AGG_RPT // _AGG_OCH

    def obuf(k):
        b = rows_a if k % 2 == 0 else rows_b
        return b.at[pl.ds(0, _AGG_OCH)]

    def isem(k):
        return sem_a if k % 2 == 0 else sem_b

    def osem(k):
        return ssem_a if k % 2 == 0 else ssem_b

    def rr(k):
        return s * _AGG_RPT + k * _AGG_OCH

    pltpu.async_copy(acc.at[pl.ds(rr(0), _AGG_OCH)], obuf(0), isem(0))
    pltpu.async_copy(acc.at[pl.ds(rr(1), _AGG_OCH)], obuf(1), isem(1))
    for k in range(nch):
        pltpu.make_async_copy(acc.at[pl.ds(rr(k), _AGG_OCH)], obuf(k),
                              isem(k)).wait()
        pltpu.async_copy(obuf(k), part_hbm.at[c, pl.ds(rr(k), _AGG_OCH)],
                         osem(k))
        if k + 2 < nch:
            pltpu.make_async_copy(obuf(k),
                                  part_hbm.at[c, pl.ds(rr(k), _AGG_OCH)],
                                  osem(k)).wait()
            pltpu.async_copy(acc.at[pl.ds(rr(k + 2), _AGG_OCH)], obuf(k + 2),
                             isem(k + 2))
    for k in (nch - 2, nch - 1):
        pltpu.make_async_copy(obuf(k), part_hbm.at[c, pl.ds(rr(k), _AGG_OCH)],
                              osem(k)).wait()


def _agg(table, edges2d):
    return pl.kernel(
        _agg_body,
        out_type=jax.ShapeDtypeStruct((NC, NP, H), jnp.float32),
        mesh=_mesh(),
        scratch_types=[
            pltpu.VMEM((_AGG_B,), jnp.int32),
            pltpu.VMEM((_AGG_B,), jnp.int32),
            pltpu.VMEM((1, _AGG_B), jnp.int32),
            pltpu.VMEM((1, _AGG_B), jnp.int32),
            pltpu.VMEM((_AGG_B, H), jnp.float32),
            pltpu.VMEM((_AGG_B, H), jnp.float32),
            pltpu.SemaphoreType.DMA,
            pltpu.SemaphoreType.DMA,
            pltpu.SemaphoreType.DMA,
            pltpu.SemaphoreType.DMA,
            pltpu.VMEM_SHARED((NP, H), jnp.float32),
        ],
        compiler_params=_SC_PARAMS,
    )(table, edges2d)


_TB = 1024


def _norm_from_deg(d):
    return jnp.where(d > 0, lax.rsqrt(jnp.maximum(d, 1e-12)), 0.0)


def _mm1_body(x_ref, w_ref, dego_ref, degi_ref, o_ref, no_ref, ni_ref):
    no = _norm_from_deg(dego_ref[...])
    ni = _norm_from_deg(degi_ref[...])
    no_ref[...] = no
    ni_ref[...] = ni
    o_ref[...] = (
        jnp.dot(x_ref[...], w_ref[...], preferred_element_type=jnp.float32)
        * no
    )


def _mm1(x, W1, dego2d, degi2d):
    return pl.pallas_call(
        _mm1_body,
        grid=(NP // _TB,),
        in_specs=[
            pl.BlockSpec((_TB, D), lambda i: (i, 0)),
            pl.BlockSpec((D, H), lambda i: (0, 0)),
            pl.BlockSpec((_TB, 1), lambda i: (i, 0)),
            pl.BlockSpec((_TB, 1), lambda i: (i, 0)),
        ],
        out_specs=[
            pl.BlockSpec((_TB, H), lambda i: (i, 0)),
            pl.BlockSpec((_TB, 1), lambda i: (i, 0)),
            pl.BlockSpec((_TB, 1), lambda i: (i, 0)),
        ],
        out_shape=[
            jax.ShapeDtypeStruct((NP, H), jnp.float32),
            jax.ShapeDtypeStruct((NP, 1), jnp.float32),
            jax.ShapeDtypeStruct((NP, 1), jnp.float32),
        ],
    )(x, W1, dego2d, degi2d)


def _mid_body(p_ref, ni_ref, b1_ref, w2_ref, no_ref, o_ref):
    sacc = p_ref[0] + p_ref[1]
    h = jnp.maximum(sacc * ni_ref[...] + b1_ref[...], 0.0)
    o_ref[...] = (
        jnp.dot(h, w2_ref[...], preferred_element_type=jnp.float32)
        * no_ref[...]
    )


def _mid(p, ni2d, b1r, W2p, no2d):
    return pl.pallas_call(
        _mid_body,
        grid=(NP // _TB,),
        in_specs=[
            pl.BlockSpec((NC, _TB, H), lambda i: (0, i, 0)),
            pl.BlockSpec((_TB, 1), lambda i: (i, 0)),
            pl.BlockSpec((1, H), lambda i: (0, 0)),
            pl.BlockSpec((H, H), lambda i: (0, 0)),
            pl.BlockSpec((_TB, 1), lambda i: (i, 0)),
        ],
        out_specs=pl.BlockSpec((_TB, H), lambda i: (i, 0)),
        out_shape=jax.ShapeDtypeStruct((NP, H), jnp.float32),
    )(p, ni2d, b1r, W2p, no2d)


def _fin_body(q_ref, ni_ref, b2_ref, o_ref):
    v = (q_ref[0] + q_ref[1]) * ni_ref[...] + b2_ref[...]
    o_ref[...] = v[:, :C]


def _fin(q, ni2d, b2r):
    nb = N // 10
    return pl.pallas_call(
        _fin_body,
        grid=(10,),
        in_specs=[
            pl.BlockSpec((NC, nb, H), lambda i: (0, i, 0)),
            pl.BlockSpec((nb, 1), lambda i: (i, 0)),
            pl.BlockSpec((1, H), lambda i: (0, 0)),
        ],
        out_specs=pl.BlockSpec((nb, C), lambda i: (i, 0)),
        out_shape=jax.ShapeDtypeStruct((N, C), jnp.float32),
    )(q, ni2d, b2r)


def kernel(in_feat, edge_index, W1, b1, W2, b2):
    edges2d = edge_index.reshape(2 * E // _AGG_B, _AGG_B)
    W2p = jnp.zeros((H, H), W2.dtype).at[:, :C].set(W2)
    b2p = jnp.zeros((1, H), b2.dtype).at[0, :C].set(b2)

    degs = _degrees(edges2d)
    xw, no2d, ni2d = _mm1(in_feat, W1, degs[:NP].reshape(NP, 1),
                          degs[NP:].reshape(NP, 1))
    p = _agg(xw, edges2d)
    t2 = _mid(p, ni2d, b1.reshape(1, H), W2p, no2d)
    q = _agg(t2, edges2d)
    return _fin(q, ni2d, b2p)
